# Initial kernel scaffold; baseline (speedup 1.0000x reference)
import jax
import jax.numpy as jnp
from jax.experimental import pallas as pl


def kernel(x, edge_index, edge_attr, W_pre, b_pre, W_post1, b_post1, W_post2, b_post2):
    raise NotImplementedError("write your pallas kernel here")


# speedup vs baseline: 1.5572x; 1.5572x over previous
import functools
import math

import jax
import jax.numpy as jnp
from jax import lax
from jax.experimental import pallas as pl
from jax.experimental.pallas import tpu as pltpu
from jax.experimental.pallas import tpu_sc as plsc

N = 10000
E = 160000
D = 256
EDGE_DIM = 16
AVG_D_LOG = float(math.log(16.0))
EPS = 1e-5
F32 = jnp.float32

NC, NS, L = 2, 16, 16
NW = NC * NS
BLK = 32
BLK_SHIFT = 5
NBLK = (N + BLK - 1) // BLK
PASSES = (NBLK + NW - 1) // NW
NPAD = NBLK * BLK
CHUNK = 2000
UNROLL = 5
CAPA = 6144
CAPB = 1024
GB = 64
DC = D // L
NEG = -3.0e38
POS = 3.0e38


def _sc_agg_body(psrc_hbm, pdst_hbm, q_hbm, src_hbm, dst_hbm,
                 out_sum, out_sq, out_mx, out_mn, out_deg,
                 src_chunk, dst_chunk, idsA, srcA, dstA,
                 srcB, qidB, lB,
                 pdst_buf, psrc_rows, q_rows,
                 acc_sum, acc_sq, acc_mx, acc_mn, deg_buf,
                 sem1, sem2):
    w = lax.axis_index("s") * NC + lax.axis_index("c")
    iota = lax.iota(jnp.int32, L)
    fz = jnp.zeros((L,), F32)
    vneg = jnp.full((L,), NEG, F32)
    vpos = jnp.full((L,), POS, F32)

    def compact(refs, vals, m, cnt, cap):
        mi = jnp.where(m, 1, 0)
        incl = plsc.cumsum(mi)
        pos = cnt + (incl - mi)
        for ref, v in zip(refs, vals):
            plsc.store_scatter(ref, [pos], v, mask=m)
        return jnp.minimum(cnt + incl[L - 1], cap)

    def chunk_body(ch, cnt):
        pltpu.sync_copy(src_hbm.at[pl.ds(ch * CHUNK, CHUNK)], src_chunk)
        pltpu.sync_copy(dst_hbm.at[pl.ds(ch * CHUNK, CHUNK)], dst_chunk)

        def vreg_body(i, cnt):
            for u in range(UNROLL):
                off = (i * UNROLL + u) * L
                d = dst_chunk[pl.ds(off, L)]
                s = src_chunk[pl.ds(off, L)]
                blk = jnp.right_shift(d, BLK_SHIFT)
                m = jnp.bitwise_and(blk, NW - 1) == w
                eid = (ch * CHUNK + off) + iota
                cnt = compact((idsA, srcA, dstA), (eid, s, d), m, cnt,
                              CAPA - L)
            return cnt

        return lax.fori_loop(0, CHUNK // (L * UNROLL), vreg_body, cnt)

    cntA = lax.fori_loop(0, E // CHUNK, chunk_body, jnp.int32(0))
    nA = (cntA + L - 1) // L

    def pass_body(p, _):
        b = p * NW + w

        @pl.when(b < NBLK)
        def _():
            base = b * BLK

            def init_body(r, _):
                for c in range(DC):
                    sl = pl.ds(c * L, L)
                    acc_sum[r, sl] = fz
                    acc_sq[r, sl] = fz
                    acc_mx[r, sl] = vneg
                    acc_mn[r, sl] = vpos
                return 0

            lax.fori_loop(0, BLK + 1, init_body, 0)
            for t in range((BLK + L) // L):
                deg_buf[pl.ds(t * L, L)] = fz

            pltpu.sync_copy(pdst_hbm.at[pl.ds(base, BLK)],
                            pdst_buf.at[pl.ds(0, BLK)])

            def sub_body(i, cnt):
                dsl = dstA[pl.ds(i * L, L)]
                ssl = srcA[pl.ds(i * L, L)]
                esl = idsA[pl.ds(i * L, L)]
                valid = (i * L + iota) < cntA
                m = (jnp.right_shift(dsl, BLK_SHIFT) == b) & valid
                return compact((srcB, qidB, lB), (ssl, esl, dsl - base),
                               m, cnt, CAPB - GB)

            cntB = lax.fori_loop(0, nA, sub_body, jnp.int32(0))

            for t in range(GB // L):
                lB[pl.ds(cntB + t * L, L)] = jnp.full((L,), BLK, jnp.int32)
                srcB[pl.ds(cntB + t * L, L)] = iota
                qidB[pl.ds(cntB + t * L, L)] = iota

            nb = (cntB + GB - 1) // GB

            def batch_body(t, _):
                i0 = t * GB
                cp1 = pltpu.async_copy(psrc_hbm.at[srcB.at[pl.ds(i0, GB)]],
                                       psrc_rows, sem1)
                cp2 = pltpu.async_copy(q_hbm.at[qidB.at[pl.ds(i0, GB)]],
                                       q_rows, sem2)
                cp1.wait()
                cp2.wait()

                def edge_grp(jj, _):
                    lvec = lB[pl.ds(i0 + jj * L, L)]
                    for j in range(L):
                        l = lvec[j]
                        slot = lax.bitwise_and(l, -L)
                        onehot = jnp.where((slot + iota) == l, 1.0, 0.0)
                        plsc.addupdate(deg_buf.at[pl.ds(slot, L)], onehot)
                        row = jj * L + j
                        for c in range(DC):
                            sl = pl.ds(c * L, L)
                            e = jnp.maximum(
                                psrc_rows[row, sl] + q_rows[row, sl]
                                + pdst_buf[l, sl], 0.0)
                            plsc.addupdate(acc_sum.at[l, sl], e)
                            plsc.addupdate(acc_sq.at[l, sl], e * e)
                            acc_mx[l, sl] = jnp.maximum(acc_mx[l, sl], e)
                            acc_mn[l, sl] = jnp.minimum(acc_mn[l, sl], e)
                    return 0

                lax.fori_loop(0, GB // L, edge_grp, 0)
                return 0

            lax.fori_loop(0, nb, batch_body, 0)

            pltpu.sync_copy(acc_sum.at[pl.ds(0, BLK)],
                            out_sum.at[pl.ds(base, BLK)])
            pltpu.sync_copy(acc_sq.at[pl.ds(0, BLK)],
                            out_sq.at[pl.ds(base, BLK)])
            pltpu.sync_copy(acc_mx.at[pl.ds(0, BLK)],
                            out_mx.at[pl.ds(base, BLK)])
            pltpu.sync_copy(acc_mn.at[pl.ds(0, BLK)],
                            out_mn.at[pl.ds(base, BLK)])
            pltpu.sync_copy(deg_buf.at[pl.ds(0, BLK)],
                            out_deg.at[pl.ds(base, BLK)])

        return 0

    lax.fori_loop(0, PASSES, pass_body, 0)


def _sc_aggregate(psrc, pdst_pad, q, src, dst):
    mesh = plsc.VectorSubcoreMesh(core_axis_name="c", subcore_axis_name="s",
                                  num_cores=NC, num_subcores=NS)
    f = pl.kernel(
        _sc_agg_body,
        out_type=(
            jax.ShapeDtypeStruct((NPAD, D), F32),
            jax.ShapeDtypeStruct((NPAD, D), F32),
            jax.ShapeDtypeStruct((NPAD, D), F32),
            jax.ShapeDtypeStruct((NPAD, D), F32),
            jax.ShapeDtypeStruct((NPAD,), F32),
        ),
        mesh=mesh,
        compiler_params=pltpu.CompilerParams(needs_layout_passes=False),
        scratch_types=[
            pltpu.VMEM((CHUNK,), jnp.int32),
            pltpu.VMEM((CHUNK,), jnp.int32),
            pltpu.VMEM((CAPA,), jnp.int32),
            pltpu.VMEM((CAPA,), jnp.int32),
            pltpu.VMEM((CAPA,), jnp.int32),
            pltpu.VMEM((CAPB,), jnp.int32),
            pltpu.VMEM((CAPB,), jnp.int32),
            pltpu.VMEM((CAPB,), jnp.int32),
            pltpu.VMEM((BLK + 1, D), F32),
            pltpu.VMEM((GB, D), F32),
            pltpu.VMEM((GB, D), F32),
            pltpu.VMEM((BLK + 1, D), F32),
            pltpu.VMEM((BLK + 1, D), F32),
            pltpu.VMEM((BLK + 1, D), F32),
            pltpu.VMEM((BLK + 1, D), F32),
            pltpu.VMEM((BLK + L, ), F32),
            pltpu.SemaphoreType.DMA,
            pltpu.SemaphoreType.DMA,
        ],
    )
    return f(psrc, pdst_pad, q, src, dst)


TM = 400
TE = 3200


def _pre_body(xr, w1r, w2r, o1r, o2r):
    xv = xr[...]
    o1r[...] = jnp.dot(xv, w1r[...], preferred_element_type=F32)
    o2r[...] = jnp.dot(xv, w2r[...], preferred_element_type=F32)


def _edge_body(ar, wr, br, qr):
    qr[...] = jnp.dot(ar[...], wr[...], preferred_element_type=F32) + br[...]


def _post_body(xr, sr, qr, mxr, mnr, degr, w1r, b1r, w2r, b2r, outr):
    deg = degr[...]
    degc = jnp.maximum(deg, 1.0)
    inv = 1.0 / degc
    has = deg > 0.0
    logd = jnp.log(deg + 1.0)
    amp = logd * (1.0 / AVG_D_LOG)
    att = AVG_D_LOG / jnp.where(logd > 0.0, logd, 1.0)

    mean = sr[...] * inv
    msq = qr[...] * inv
    var = jnp.maximum(msq - mean * mean, 0.0)
    std = jnp.sqrt(var + EPS)
    zero = jnp.zeros((), F32)
    aggs = (jnp.where(has, mean, zero),
            jnp.where(has, mxr[...], zero),
            jnp.where(has, mnr[...], zero),
            jnp.where(has, std, zero))

    xv = xr[...]
    acc = jnp.dot(xv, w1r[0:D, :], preferred_element_type=F32)
    for j, z in enumerate(aggs):
        y_id = jnp.dot(z, w1r[(1 + j) * D:(2 + j) * D, :],
                       preferred_element_type=F32)
        y_amp = jnp.dot(z, w1r[(5 + j) * D:(6 + j) * D, :],
                        preferred_element_type=F32)
        y_att = jnp.dot(z, w1r[(9 + j) * D:(10 + j) * D, :],
                        preferred_element_type=F32)
        acc = acc + y_id + amp * y_amp + att * y_att
    hidden = jnp.maximum(acc + b1r[...], 0.0)
    outr[...] = (jnp.dot(hidden, w2r[...], preferred_element_type=F32)
                 + b2r[...] + xv)


def _pre_mm(x, w1, w2):
    return pl.pallas_call(
        _pre_body,
        grid=(N // TM,),
        in_specs=[
            pl.BlockSpec((TM, D), lambda i: (i, 0)),
            pl.BlockSpec((D, D), lambda i: (0, 0)),
            pl.BlockSpec((D, D), lambda i: (0, 0)),
        ],
        out_specs=[
            pl.BlockSpec((TM, D), lambda i: (i, 0)),
            pl.BlockSpec((TM, D), lambda i: (i, 0)),
        ],
        out_shape=[
            jax.ShapeDtypeStruct((N, D), F32),
            jax.ShapeDtypeStruct((N, D), F32),
        ],
    )(x, w1, w2)


def _edge_mm(edge_attr, w3, b):
    return pl.pallas_call(
        _edge_body,
        grid=(E // TE,),
        in_specs=[
            pl.BlockSpec((TE, EDGE_DIM), lambda i: (i, 0)),
            pl.BlockSpec((EDGE_DIM, D), lambda i: (0, 0)),
            pl.BlockSpec((1, D), lambda i: (0, 0)),
        ],
        out_specs=pl.BlockSpec((TE, D), lambda i: (i, 0)),
        out_shape=jax.ShapeDtypeStruct((E, D), F32),
    )(edge_attr, w3, b)


def _post_mm(x, s, sq, mx, mn, deg2d, w1, b1, w2, b2):
    return pl.pallas_call(
        _post_body,
        grid=(N // TM,),
        in_specs=[
            pl.BlockSpec((TM, D), lambda i: (i, 0)),
            pl.BlockSpec((TM, D), lambda i: (i, 0)),
            pl.BlockSpec((TM, D), lambda i: (i, 0)),
            pl.BlockSpec((TM, D), lambda i: (i, 0)),
            pl.BlockSpec((TM, D), lambda i: (i, 0)),
            pl.BlockSpec((TM, 1), lambda i: (i, 0)),
            pl.BlockSpec((13 * D, D), lambda i: (0, 0)),
            pl.BlockSpec((1, D), lambda i: (0, 0)),
            pl.BlockSpec((D, D), lambda i: (0, 0)),
            pl.BlockSpec((1, D), lambda i: (0, 0)),
        ],
        out_specs=pl.BlockSpec((TM, D), lambda i: (i, 0)),
        out_shape=jax.ShapeDtypeStruct((N, D), F32),
    )(x, s, sq, mx, mn, deg2d, w1, b1, w2, b2)


def kernel(x, edge_index, edge_attr, W_pre, b_pre, W_post1, b_post1,
           W_post2, b_post2):
    src = edge_index[0]
    dst = edge_index[1]
    psrc, pdst = _pre_mm(x, W_pre[0:D, :], W_pre[D:2 * D, :])
    q = _edge_mm(edge_attr, W_pre[2 * D:, :], b_pre.reshape(1, D))
    pdst_pad = jnp.pad(pdst, ((0, NPAD - N), (0, 0)))
    s, sq, mx, mn, deg = _sc_aggregate(psrc, pdst_pad, q, src, dst)
    out = _post_mm(x, s[:N], sq[:N], mx[:N], mn[:N],
                   deg[:N].reshape(N, 1), W_post1,
                   b_post1.reshape(1, D), W_post2, b_post2.reshape(1, D))
    return out


# speedup vs baseline: 4.0613x; 2.6081x over previous
import functools
import math

import jax
import jax.numpy as jnp
from jax import lax
from jax.experimental import pallas as pl
from jax.experimental.pallas import tpu as pltpu
from jax.experimental.pallas import tpu_sc as plsc

N = 10000
E = 160000
D = 256
EDGE_DIM = 16
AVG_D_LOG = float(math.log(16.0))
EPS = 1e-5
F32 = jnp.float32

NC, NS, L = 2, 16, 16
NW = NC * NS
BLK = 32
BLK_SHIFT = 5
NBLK = (N + BLK - 1) // BLK
PASSES = (NBLK + NW - 1) // NW
NPAD = NBLK * BLK
CHUNK = 2000
UNROLL = 5
CAPA = 6144
CAPB = 1024
GB = 64
DC = D // L
NEG = -3.0e38
POS = 3.0e38


def _sc_agg_body(psrc_hbm, pdst_hbm, q_hbm, src_hbm, dst_hbm,
                 out_sum, out_sq, out_mx, out_mn, out_deg,
                 src_chunk, dst_chunk, idsA, srcA, dstA,
                 srcB, qidB, lB, srcS, qidS, lS,
                 pdst_buf, psrc_rows, q_rows,
                 acc_sum, acc_sq, acc_mx, acc_mn, deg_buf,
                 hist, offs0, offs1,
                 sem1, sem2):
    w = lax.axis_index("s") * NC + lax.axis_index("c")
    iota = lax.iota(jnp.int32, L)
    fz = jnp.zeros((L,), F32)
    vneg = jnp.full((L,), NEG, F32)
    vpos = jnp.full((L,), POS, F32)

    def compact(refs, vals, m, cnt, cap):
        mi = jnp.where(m, 1, 0)
        incl = plsc.cumsum(mi)
        pos = cnt + (incl - mi)
        for ref, v in zip(refs, vals):
            plsc.store_scatter(ref, [pos], v, mask=m)
        return jnp.minimum(cnt + incl[L - 1], cap)

    def chunk_body(ch, cnt):
        pltpu.sync_copy(src_hbm.at[pl.ds(ch * CHUNK, CHUNK)], src_chunk)
        pltpu.sync_copy(dst_hbm.at[pl.ds(ch * CHUNK, CHUNK)], dst_chunk)

        def vreg_body(i, cnt):
            for u in range(UNROLL):
                off = (i * UNROLL + u) * L
                d = dst_chunk[pl.ds(off, L)]
                s = src_chunk[pl.ds(off, L)]
                blk = jnp.right_shift(d, BLK_SHIFT)
                m = jnp.bitwise_and(blk, NW - 1) == w
                eid = (ch * CHUNK + off) + iota
                cnt = compact((idsA, srcA, dstA), (eid, s, d), m, cnt,
                              CAPA - L)
            return cnt

        return lax.fori_loop(0, CHUNK // (L * UNROLL), vreg_body, cnt)

    cntA = lax.fori_loop(0, E // CHUNK, chunk_body, jnp.int32(0))
    nA = (cntA + L - 1) // L

    def pass_body(p, _):
        b = p * NW + w

        @pl.when(b < NBLK)
        def _():
            base = b * BLK

            def init_body(r, _):
                for c in range(DC):
                    sl = pl.ds(c * L, L)
                    acc_sum[r, sl] = fz
                    acc_sq[r, sl] = fz
                    acc_mx[r, sl] = vneg
                    acc_mn[r, sl] = vpos
                return 0

            lax.fori_loop(0, BLK + 1, init_body, 0)

            pltpu.sync_copy(pdst_hbm.at[pl.ds(base, BLK)],
                            pdst_buf.at[pl.ds(0, BLK)])

            def sub_body(i, cnt):
                dsl = dstA[pl.ds(i * L, L)]
                ssl = srcA[pl.ds(i * L, L)]
                esl = idsA[pl.ds(i * L, L)]
                valid = (i * L + iota) < cntA
                m = (jnp.right_shift(dsl, BLK_SHIFT) == b) & valid
                return compact((srcB, qidB, lB), (ssl, esl, dsl - base),
                               m, cnt, CAPB - GB)

            cntB = lax.fori_loop(0, nA, sub_body, jnp.int32(0))

            for t in range(GB // L):
                lB[pl.ds(cntB + t * L, L)] = jnp.full((L,), BLK, jnp.int32)
                srcB[pl.ds(cntB + t * L, L)] = iota
                qidB[pl.ds(cntB + t * L, L)] = iota

            nb = (cntB + GB - 1) // GB
            nv = nb * (GB // L)

            for i in range(BLK + 2):
                hist[i] = jnp.int32(0)

            def hist_body(v, _):
                lvec = lB[pl.ds(v * L, L)]
                for j in range(L):
                    l = lvec[j]
                    hist[l] = hist[l] + 1
                return 0

            lax.fori_loop(0, nv, hist_body, 0)

            run = jnp.int32(0)
            for l in range(BLK + 2):
                offs0[l] = run
                offs1[l] = run
                if l <= BLK:
                    run = run + hist[l]

            def perm_body(v, _):
                lvec = lB[pl.ds(v * L, L)]
                svec = srcB[pl.ds(v * L, L)]
                qvec = qidB[pl.ds(v * L, L)]
                posv = jnp.zeros((L,), jnp.int32)
                for j in range(L):
                    l = lvec[j]
                    pos = offs1[l]
                    offs1[l] = pos + 1
                    posv = jnp.where(iota == j, pos, posv)
                plsc.store_scatter(srcS, [posv], svec)
                plsc.store_scatter(qidS, [posv], qvec)
                plsc.store_scatter(lS, [posv], lvec)
                return 0

            lax.fori_loop(0, nv, perm_body, 0)

            for vi in range(BLK // L):
                dv = jnp.zeros((L,), F32)
                for j in range(L):
                    dv = jnp.where(iota == j,
                                   hist[vi * L + j].astype(F32), dv)
                deg_buf[pl.ds(vi * L, L)] = dv

            def batch_body(t, _):
                i0 = t * GB
                cp1 = pltpu.async_copy(psrc_hbm.at[srcS.at[pl.ds(i0, GB)]],
                                       psrc_rows, sem1)
                cp2 = pltpu.async_copy(q_hbm.at[qidS.at[pl.ds(i0, GB)]],
                                       q_rows, sem2)
                cp1.wait()
                cp2.wait()
                l_lo = lS[pl.ds(i0, L)][0]
                l_hi = lS[pl.ds(i0 + GB - L, L)][L - 1]

                def node_body(l, _):
                    o0 = jnp.maximum(offs0[l], i0)
                    o1 = jnp.minimum(offs0[l + 1], i0 + GB)

                    @pl.when(o1 > o0)
                    def _():
                        for g in range(4):
                            pd = [pdst_buf[l, pl.ds((g * 4 + cc) * L, L)]
                                  for cc in range(4)]
                            zz = jnp.zeros((L,), F32)
                            init = ([zz] * 4 + [zz] * 4
                                    + [vneg] * 4 + [vpos] * 4)

                            def jbody(j, carry):
                                r = j - i0
                                out = list(carry)
                                for cc in range(4):
                                    sl = pl.ds((g * 4 + cc) * L, L)
                                    e = jnp.maximum(
                                        psrc_rows[r, sl] + q_rows[r, sl]
                                        + pd[cc], 0.0)
                                    out[cc] = carry[cc] + e
                                    out[4 + cc] = carry[4 + cc] + e * e
                                    out[8 + cc] = jnp.maximum(
                                        carry[8 + cc], e)
                                    out[12 + cc] = jnp.minimum(
                                        carry[12 + cc], e)
                                return tuple(out)

                            res = lax.fori_loop(o0, o1, jbody, tuple(init))
                            for cc in range(4):
                                sl = pl.ds((g * 4 + cc) * L, L)
                                plsc.addupdate(acc_sum.at[l, sl], res[cc])
                                plsc.addupdate(acc_sq.at[l, sl],
                                               res[4 + cc])
                                acc_mx[l, sl] = jnp.maximum(
                                    acc_mx[l, sl], res[8 + cc])
                                acc_mn[l, sl] = jnp.minimum(
                                    acc_mn[l, sl], res[12 + cc])
                    return 0

                lax.fori_loop(l_lo, l_hi + 1, node_body, 0)
                return 0

            lax.fori_loop(0, nb, batch_body, 0)

            pltpu.sync_copy(acc_sum.at[pl.ds(0, BLK)],
                            out_sum.at[pl.ds(base, BLK)])
            pltpu.sync_copy(acc_sq.at[pl.ds(0, BLK)],
                            out_sq.at[pl.ds(base, BLK)])
            pltpu.sync_copy(acc_mx.at[pl.ds(0, BLK)],
                            out_mx.at[pl.ds(base, BLK)])
            pltpu.sync_copy(acc_mn.at[pl.ds(0, BLK)],
                            out_mn.at[pl.ds(base, BLK)])
            pltpu.sync_copy(deg_buf.at[pl.ds(0, BLK)],
                            out_deg.at[pl.ds(base, BLK)])

        return 0

    lax.fori_loop(0, PASSES, pass_body, 0)


def _sc_aggregate(psrc, pdst_pad, q, src, dst):
    mesh = plsc.VectorSubcoreMesh(core_axis_name="c", subcore_axis_name="s",
                                  num_cores=NC, num_subcores=NS)
    f = pl.kernel(
        _sc_agg_body,
        out_type=(
            jax.ShapeDtypeStruct((NPAD, D), F32),
            jax.ShapeDtypeStruct((NPAD, D), F32),
            jax.ShapeDtypeStruct((NPAD, D), F32),
            jax.ShapeDtypeStruct((NPAD, D), F32),
            jax.ShapeDtypeStruct((NPAD,), F32),
        ),
        mesh=mesh,
        compiler_params=pltpu.CompilerParams(needs_layout_passes=False),
        scratch_types=[
            pltpu.VMEM((CHUNK,), jnp.int32),
            pltpu.VMEM((CHUNK,), jnp.int32),
            pltpu.VMEM((CAPA,), jnp.int32),
            pltpu.VMEM((CAPA,), jnp.int32),
            pltpu.VMEM((CAPA,), jnp.int32),
            pltpu.VMEM((CAPB,), jnp.int32),
            pltpu.VMEM((CAPB,), jnp.int32),
            pltpu.VMEM((CAPB,), jnp.int32),
            pltpu.VMEM((CAPB,), jnp.int32),
            pltpu.VMEM((CAPB,), jnp.int32),
            pltpu.VMEM((CAPB,), jnp.int32),
            pltpu.VMEM((BLK + 1, D), F32),
            pltpu.VMEM((GB, D), F32),
            pltpu.VMEM((GB, D), F32),
            pltpu.VMEM((BLK + 1, D), F32),
            pltpu.VMEM((BLK + 1, D), F32),
            pltpu.VMEM((BLK + 1, D), F32),
            pltpu.VMEM((BLK + 1, D), F32),
            pltpu.VMEM((BLK,), F32),
            pltpu.SMEM((BLK + 2,), jnp.int32),
            pltpu.SMEM((BLK + 2,), jnp.int32),
            pltpu.SMEM((BLK + 2,), jnp.int32),
            pltpu.SemaphoreType.DMA,
            pltpu.SemaphoreType.DMA,
        ],
    )
    return f(psrc, pdst_pad, q, src, dst)


TM = 400
TE = 3200


def _pre_body(xr, w1r, w2r, o1r, o2r):
    xv = xr[...]
    o1r[...] = jnp.dot(xv, w1r[...], preferred_element_type=F32)
    o2r[...] = jnp.dot(xv, w2r[...], preferred_element_type=F32)


def _edge_body(ar, wr, br, qr):
    qr[...] = jnp.dot(ar[...], wr[...], preferred_element_type=F32) + br[...]


def _post_body(xr, sr, qr, mxr, mnr, degr, w1r, b1r, w2r, b2r, outr):
    deg = degr[...]
    degc = jnp.maximum(deg, 1.0)
    inv = 1.0 / degc
    has = deg > 0.0
    logd = jnp.log(deg + 1.0)
    amp = logd * (1.0 / AVG_D_LOG)
    att = AVG_D_LOG / jnp.where(logd > 0.0, logd, 1.0)

    mean = sr[...] * inv
    msq = qr[...] * inv
    var = jnp.maximum(msq - mean * mean, 0.0)
    std = jnp.sqrt(var + EPS)
    zero = jnp.zeros((), F32)
    aggs = (jnp.where(has, mean, zero),
            jnp.where(has, mxr[...], zero),
            jnp.where(has, mnr[...], zero),
            jnp.where(has, std, zero))

    xv = xr[...]
    acc = jnp.dot(xv, w1r[0:D, :], preferred_element_type=F32)
    for j, z in enumerate(aggs):
        y_id = jnp.dot(z, w1r[(1 + j) * D:(2 + j) * D, :],
                       preferred_element_type=F32)
        y_amp = jnp.dot(z, w1r[(5 + j) * D:(6 + j) * D, :],
                        preferred_element_type=F32)
        y_att = jnp.dot(z, w1r[(9 + j) * D:(10 + j) * D, :],
                        preferred_element_type=F32)
        acc = acc + y_id + amp * y_amp + att * y_att
    hidden = jnp.maximum(acc + b1r[...], 0.0)
    outr[...] = (jnp.dot(hidden, w2r[...], preferred_element_type=F32)
                 + b2r[...] + xv)


def _pre_mm(x, w1, w2):
    return pl.pallas_call(
        _pre_body,
        grid=(N // TM,),
        in_specs=[
            pl.BlockSpec((TM, D), lambda i: (i, 0)),
            pl.BlockSpec((D, D), lambda i: (0, 0)),
            pl.BlockSpec((D, D), lambda i: (0, 0)),
        ],
        out_specs=[
            pl.BlockSpec((TM, D), lambda i: (i, 0)),
            pl.BlockSpec((TM, D), lambda i: (i, 0)),
        ],
        out_shape=[
            jax.ShapeDtypeStruct((N, D), F32),
            jax.ShapeDtypeStruct((N, D), F32),
        ],
    )(x, w1, w2)


def _edge_mm(edge_attr, w3, b):
    return pl.pallas_call(
        _edge_body,
        grid=(E // TE,),
        in_specs=[
            pl.BlockSpec((TE, EDGE_DIM), lambda i: (i, 0)),
            pl.BlockSpec((EDGE_DIM, D), lambda i: (0, 0)),
            pl.BlockSpec((1, D), lambda i: (0, 0)),
        ],
        out_specs=pl.BlockSpec((TE, D), lambda i: (i, 0)),
        out_shape=jax.ShapeDtypeStruct((E, D), F32),
    )(edge_attr, w3, b)


def _post_mm(x, s, sq, mx, mn, deg2d, w1, b1, w2, b2):
    return pl.pallas_call(
        _post_body,
        grid=(N // TM,),
        in_specs=[
            pl.BlockSpec((TM, D), lambda i: (i, 0)),
            pl.BlockSpec((TM, D), lambda i: (i, 0)),
            pl.BlockSpec((TM, D), lambda i: (i, 0)),
            pl.BlockSpec((TM, D), lambda i: (i, 0)),
            pl.BlockSpec((TM, D), lambda i: (i, 0)),
            pl.BlockSpec((TM, 1), lambda i: (i, 0)),
            pl.BlockSpec((13 * D, D), lambda i: (0, 0)),
            pl.BlockSpec((1, D), lambda i: (0, 0)),
            pl.BlockSpec((D, D), lambda i: (0, 0)),
            pl.BlockSpec((1, D), lambda i: (0, 0)),
        ],
        out_specs=pl.BlockSpec((TM, D), lambda i: (i, 0)),
        out_shape=jax.ShapeDtypeStruct((N, D), F32),
    )(x, s, sq, mx, mn, deg2d, w1, b1, w2, b2)


def kernel(x, edge_index, edge_attr, W_pre, b_pre, W_post1, b_post1,
           W_post2, b_post2):
    src = edge_index[0]
    dst = edge_index[1]
    psrc, pdst = _pre_mm(x, W_pre[0:D, :], W_pre[D:2 * D, :])
    q = _edge_mm(edge_attr, W_pre[2 * D:, :], b_pre.reshape(1, D))
    pdst_pad = jnp.pad(pdst, ((0, NPAD - N), (0, 0)))
    s, sq, mx, mn, deg = _sc_aggregate(psrc, pdst_pad, q, src, dst)
    out = _post_mm(x, s[:N], sq[:N], mx[:N], mn[:N],
                   deg[:N].reshape(N, 1), W_post1,
                   b_post1.reshape(1, D), W_post2, b_post2.reshape(1, D))
    return out


# speedup vs baseline: 4.8756x; 1.2005x over previous
import functools
import math

import jax
import jax.numpy as jnp
from jax import lax
from jax.experimental import pallas as pl
from jax.experimental.pallas import tpu as pltpu
from jax.experimental.pallas import tpu_sc as plsc

N = 10000
E = 160000
D = 256
EDGE_DIM = 16
AVG_D_LOG = float(math.log(16.0))
EPS = 1e-5
F32 = jnp.float32

NC, NS, L = 2, 16, 16
NW = NC * NS
BLK = 32
BLK_SHIFT = 5
NBLK = (N + BLK - 1) // BLK
PASSES = (NBLK + NW - 1) // NW
NPAD = NBLK * BLK
CHUNK = 3200
UNROLL = 5
CAPA = 5888
CAPB = 1024
GB = 48
DC = D // L
NEG = -3.0e38
POS = 3.0e38


def _sc_agg_body(psrc_hbm, pdst_hbm, q_hbm, src_hbm, dst_hbm,
                 out_sum, out_sq, out_mx, out_mn, out_deg,
                 src_chunk, dst_chunk, idsA, srcA, dstA,
                 srcB, qidB, lB, srcS, qidS, lS,
                 pdst_buf, psrc0, psrc1, qr0, qr1,
                 acc_sum, acc_sq, acc_mx, acc_mn, deg_buf,
                 hist, offs0, offs1,
                 semp0, semp1, semq0, semq1):
    w = lax.axis_index("s") * NC + lax.axis_index("c")
    iota = lax.iota(jnp.int32, L)
    fz = jnp.zeros((L,), F32)
    vneg = jnp.full((L,), NEG, F32)
    vpos = jnp.full((L,), POS, F32)

    def compact(refs, vals, m, cnt, cap):
        mi = jnp.where(m, 1, 0)
        incl = plsc.cumsum(mi)
        pos = cnt + (incl - mi)
        for ref, v in zip(refs, vals):
            plsc.store_scatter(ref, [pos], v, mask=m)
        return jnp.minimum(cnt + incl[L - 1], cap)

    def chunk_body(ch, cnt):
        pltpu.sync_copy(src_hbm.at[pl.ds(ch * CHUNK, CHUNK)], src_chunk)
        pltpu.sync_copy(dst_hbm.at[pl.ds(ch * CHUNK, CHUNK)], dst_chunk)

        def vreg_body(i, cnt):
            for u in range(UNROLL):
                off = (i * UNROLL + u) * L
                d = dst_chunk[pl.ds(off, L)]
                s = src_chunk[pl.ds(off, L)]
                blk = jnp.right_shift(d, BLK_SHIFT)
                m = jnp.bitwise_and(blk, NW - 1) == w
                eid = (ch * CHUNK + off) + iota
                cnt = compact((idsA, srcA, dstA), (eid, s, d), m, cnt,
                              CAPA - L)
            return cnt

        return lax.fori_loop(0, CHUNK // (L * UNROLL), vreg_body, cnt)

    cntA = lax.fori_loop(0, E // CHUNK, chunk_body, jnp.int32(0))
    nA = (cntA + L - 1) // L

    def pass_body(p, _):
        b = p * NW + w

        @pl.when(b < NBLK)
        def _():
            base = b * BLK

            def init_body(r, _):
                for c in range(DC):
                    sl = pl.ds(c * L, L)
                    acc_sum[r, sl] = fz
                    acc_sq[r, sl] = fz
                    acc_mx[r, sl] = vneg
                    acc_mn[r, sl] = vpos
                return 0

            lax.fori_loop(0, BLK + 1, init_body, 0)

            pltpu.sync_copy(pdst_hbm.at[pl.ds(base, BLK)],
                            pdst_buf.at[pl.ds(0, BLK)])

            def sub_body(i, cnt):
                dsl = dstA[pl.ds(i * L, L)]
                ssl = srcA[pl.ds(i * L, L)]
                esl = idsA[pl.ds(i * L, L)]
                valid = (i * L + iota) < cntA
                m = (jnp.right_shift(dsl, BLK_SHIFT) == b) & valid
                return compact((srcB, qidB, lB), (ssl, esl, dsl - base),
                               m, cnt, CAPB - GB)

            cntB = lax.fori_loop(0, nA, sub_body, jnp.int32(0))

            for t in range(GB // L):
                lB[pl.ds(cntB + t * L, L)] = jnp.full((L,), BLK, jnp.int32)
                srcB[pl.ds(cntB + t * L, L)] = iota
                qidB[pl.ds(cntB + t * L, L)] = iota

            nb = (cntB + GB - 1) // GB
            nv = nb * (GB // L)

            for i in range(BLK + 2):
                hist[i] = jnp.int32(0)

            def hist_body(v, _):
                lvec = lB[pl.ds(v * L, L)]
                for j in range(L):
                    l = lvec[j]
                    hist[l] = hist[l] + 1
                return 0

            lax.fori_loop(0, nv, hist_body, 0)

            run = jnp.int32(0)
            for l in range(BLK + 2):
                offs0[l] = run
                offs1[l] = run
                if l <= BLK:
                    run = run + hist[l]

            def perm_body(v, _):
                lvec = lB[pl.ds(v * L, L)]
                svec = srcB[pl.ds(v * L, L)]
                qvec = qidB[pl.ds(v * L, L)]
                posv = jnp.zeros((L,), jnp.int32)
                for j in range(L):
                    l = lvec[j]
                    pos = offs1[l]
                    offs1[l] = pos + 1
                    posv = jnp.where(iota == j, pos, posv)
                plsc.store_scatter(srcS, [posv], svec)
                plsc.store_scatter(qidS, [posv], qvec)
                plsc.store_scatter(lS, [posv], lvec)
                return 0

            lax.fori_loop(0, nv, perm_body, 0)

            for vi in range(BLK // L):
                dv = jnp.zeros((L,), F32)
                for j in range(L):
                    dv = jnp.where(iota == j,
                                   hist[vi * L + j].astype(F32), dv)
                deg_buf[pl.ds(vi * L, L)] = dv

            rows = ((psrc0, qr0, semp0, semq0), (psrc1, qr1, semp1, semq1))

            def start(t, slot):
                pr, qr, sp, sq_ = rows[slot]
                pltpu.async_copy(psrc_hbm.at[srcS.at[pl.ds(t * GB, GB)]],
                                 pr, sp)
                pltpu.async_copy(q_hbm.at[qidS.at[pl.ds(t * GB, GB)]],
                                 qr, sq_)

            @pl.when(nb > 0)
            def _():
                start(0, 0)

            def batch_body(t, slot):
                pr, qr, sp, sq_ = rows[slot]
                i0 = t * GB
                pltpu.make_async_copy(
                    psrc_hbm.at[srcS.at[pl.ds(i0, GB)]], pr, sp).wait()
                pltpu.make_async_copy(
                    q_hbm.at[qidS.at[pl.ds(i0, GB)]], qr, sq_).wait()

                @pl.when(t + 1 < nb)
                def _():
                    start(t + 1, 1 - slot)

                psrc_rows, q_rows = pr, qr
                l_lo = lS[pl.ds(i0, L)][0]
                l_hi = lS[pl.ds(i0 + GB - L, L)][L - 1]

                def node_body(l, _):
                    o0 = jnp.maximum(offs0[l], i0)
                    o1 = jnp.minimum(offs0[l + 1], i0 + GB)

                    @pl.when(o1 > o0)
                    def _():
                        for g in range(4):
                            pd = [pdst_buf[l, pl.ds((g * 4 + cc) * L, L)]
                                  for cc in range(4)]
                            zz = jnp.zeros((L,), F32)
                            init = ([zz] * 4 + [zz] * 4
                                    + [vneg] * 4 + [vpos] * 4)

                            def jbody(j, carry):
                                r = j - i0
                                out = list(carry)
                                for cc in range(4):
                                    sl = pl.ds((g * 4 + cc) * L, L)
                                    e = jnp.maximum(
                                        psrc_rows[r, sl] + q_rows[r, sl]
                                        + pd[cc], 0.0)
                                    out[cc] = carry[cc] + e
                                    out[4 + cc] = carry[4 + cc] + e * e
                                    out[8 + cc] = jnp.maximum(
                                        carry[8 + cc], e)
                                    out[12 + cc] = jnp.minimum(
                                        carry[12 + cc], e)
                                return tuple(out)

                            res = lax.fori_loop(o0, o1, jbody, tuple(init))
                            for cc in range(4):
                                sl = pl.ds((g * 4 + cc) * L, L)
                                plsc.addupdate(acc_sum.at[l, sl], res[cc])
                                plsc.addupdate(acc_sq.at[l, sl],
                                               res[4 + cc])
                                acc_mx[l, sl] = jnp.maximum(
                                    acc_mx[l, sl], res[8 + cc])
                                acc_mn[l, sl] = jnp.minimum(
                                    acc_mn[l, sl], res[12 + cc])
                    return 0

                lax.fori_loop(l_lo, l_hi + 1, node_body, 0)

            def pair_body(gp, _):
                for s2 in range(2):
                    t = gp * 2 + s2

                    @pl.when(t < nb)
                    def _():
                        batch_body(t, s2)
                return 0

            lax.fori_loop(0, (nb + 1) // 2, pair_body, 0)

            pltpu.sync_copy(acc_sum.at[pl.ds(0, BLK)],
                            out_sum.at[pl.ds(base, BLK)])
            pltpu.sync_copy(acc_sq.at[pl.ds(0, BLK)],
                            out_sq.at[pl.ds(base, BLK)])
            pltpu.sync_copy(acc_mx.at[pl.ds(0, BLK)],
                            out_mx.at[pl.ds(base, BLK)])
            pltpu.sync_copy(acc_mn.at[pl.ds(0, BLK)],
                            out_mn.at[pl.ds(base, BLK)])
            pltpu.sync_copy(deg_buf.at[pl.ds(0, BLK)],
                            out_deg.at[pl.ds(base, BLK)])

        return 0

    lax.fori_loop(0, PASSES, pass_body, 0)


def _sc_aggregate(psrc, pdst_pad, q, src, dst):
    mesh = plsc.VectorSubcoreMesh(core_axis_name="c", subcore_axis_name="s",
                                  num_cores=NC, num_subcores=NS)
    f = pl.kernel(
        _sc_agg_body,
        out_type=(
            jax.ShapeDtypeStruct((NPAD, D), F32),
            jax.ShapeDtypeStruct((NPAD, D), F32),
            jax.ShapeDtypeStruct((NPAD, D), F32),
            jax.ShapeDtypeStruct((NPAD, D), F32),
            jax.ShapeDtypeStruct((NPAD,), F32),
        ),
        mesh=mesh,
        compiler_params=pltpu.CompilerParams(needs_layout_passes=False),
        scratch_types=[
            pltpu.VMEM((CHUNK,), jnp.int32),
            pltpu.VMEM((CHUNK,), jnp.int32),
            pltpu.VMEM((CAPA,), jnp.int32),
            pltpu.VMEM((CAPA,), jnp.int32),
            pltpu.VMEM((CAPA,), jnp.int32),
            pltpu.VMEM((CAPB,), jnp.int32),
            pltpu.VMEM((CAPB,), jnp.int32),
            pltpu.VMEM((CAPB,), jnp.int32),
            pltpu.VMEM((CAPB,), jnp.int32),
            pltpu.VMEM((CAPB,), jnp.int32),
            pltpu.VMEM((CAPB,), jnp.int32),
            pltpu.VMEM((BLK + 1, D), F32),
            pltpu.VMEM((GB, D), F32),
            pltpu.VMEM((GB, D), F32),
            pltpu.VMEM((GB, D), F32),
            pltpu.VMEM((GB, D), F32),
            pltpu.VMEM((BLK + 1, D), F32),
            pltpu.VMEM((BLK + 1, D), F32),
            pltpu.VMEM((BLK + 1, D), F32),
            pltpu.VMEM((BLK + 1, D), F32),
            pltpu.VMEM((BLK,), F32),
            pltpu.SMEM((BLK + 2,), jnp.int32),
            pltpu.SMEM((BLK + 2,), jnp.int32),
            pltpu.SMEM((BLK + 2,), jnp.int32),
            pltpu.SemaphoreType.DMA,
            pltpu.SemaphoreType.DMA,
            pltpu.SemaphoreType.DMA,
            pltpu.SemaphoreType.DMA,
        ],
    )
    return f(psrc, pdst_pad, q, src, dst)


TM = 400
TE = 3200


def _pre_body(xr, w1r, w2r, o1r, o2r):
    xv = xr[...]
    o1r[...] = jnp.dot(xv, w1r[...], preferred_element_type=F32)
    o2r[...] = jnp.dot(xv, w2r[...], preferred_element_type=F32)


def _edge_body(ar, wr, br, qr):
    qr[...] = jnp.dot(ar[...], wr[...], preferred_element_type=F32) + br[...]


def _post_body(xr, sr, qr, mxr, mnr, degr, w1r, b1r, w2r, b2r, outr):
    deg = degr[...]
    degc = jnp.maximum(deg, 1.0)
    inv = 1.0 / degc
    has = deg > 0.0
    logd = jnp.log(deg + 1.0)
    amp = logd * (1.0 / AVG_D_LOG)
    att = AVG_D_LOG / jnp.where(logd > 0.0, logd, 1.0)

    mean = sr[...] * inv
    msq = qr[...] * inv
    var = jnp.maximum(msq - mean * mean, 0.0)
    std = jnp.sqrt(var + EPS)
    zero = jnp.zeros((), F32)
    aggs = (jnp.where(has, mean, zero),
            jnp.where(has, mxr[...], zero),
            jnp.where(has, mnr[...], zero),
            jnp.where(has, std, zero))

    xv = xr[...]
    acc = jnp.dot(xv, w1r[0:D, :], preferred_element_type=F32)
    for j, z in enumerate(aggs):
        y_id = jnp.dot(z, w1r[(1 + j) * D:(2 + j) * D, :],
                       preferred_element_type=F32)
        y_amp = jnp.dot(z, w1r[(5 + j) * D:(6 + j) * D, :],
                        preferred_element_type=F32)
        y_att = jnp.dot(z, w1r[(9 + j) * D:(10 + j) * D, :],
                        preferred_element_type=F32)
        acc = acc + y_id + amp * y_amp + att * y_att
    hidden = jnp.maximum(acc + b1r[...], 0.0)
    outr[...] = (jnp.dot(hidden, w2r[...], preferred_element_type=F32)
                 + b2r[...] + xv)


def _pre_mm(x, w1, w2):
    return pl.pallas_call(
        _pre_body,
        grid=(N // TM,),
        in_specs=[
            pl.BlockSpec((TM, D), lambda i: (i, 0)),
            pl.BlockSpec((D, D), lambda i: (0, 0)),
            pl.BlockSpec((D, D), lambda i: (0, 0)),
        ],
        out_specs=[
            pl.BlockSpec((TM, D), lambda i: (i, 0)),
            pl.BlockSpec((TM, D), lambda i: (i, 0)),
        ],
        out_shape=[
            jax.ShapeDtypeStruct((N, D), F32),
            jax.ShapeDtypeStruct((N, D), F32),
        ],
    )(x, w1, w2)


def _edge_mm(edge_attr, w3, b):
    return pl.pallas_call(
        _edge_body,
        grid=(E // TE,),
        in_specs=[
            pl.BlockSpec((TE, EDGE_DIM), lambda i: (i, 0)),
            pl.BlockSpec((EDGE_DIM, D), lambda i: (0, 0)),
            pl.BlockSpec((1, D), lambda i: (0, 0)),
        ],
        out_specs=pl.BlockSpec((TE, D), lambda i: (i, 0)),
        out_shape=jax.ShapeDtypeStruct((E, D), F32),
    )(edge_attr, w3, b)


def _post_mm(x, s, sq, mx, mn, deg2d, w1, b1, w2, b2):
    return pl.pallas_call(
        _post_body,
        grid=(N // TM,),
        in_specs=[
            pl.BlockSpec((TM, D), lambda i: (i, 0)),
            pl.BlockSpec((TM, D), lambda i: (i, 0)),
            pl.BlockSpec((TM, D), lambda i: (i, 0)),
            pl.BlockSpec((TM, D), lambda i: (i, 0)),
            pl.BlockSpec((TM, D), lambda i: (i, 0)),
            pl.BlockSpec((TM, 1), lambda i: (i, 0)),
            pl.BlockSpec((13 * D, D), lambda i: (0, 0)),
            pl.BlockSpec((1, D), lambda i: (0, 0)),
            pl.BlockSpec((D, D), lambda i: (0, 0)),
            pl.BlockSpec((1, D), lambda i: (0, 0)),
        ],
        out_specs=pl.BlockSpec((TM, D), lambda i: (i, 0)),
        out_shape=jax.ShapeDtypeStruct((N, D), F32),
    )(x, s, sq, mx, mn, deg2d, w1, b1, w2, b2)


def kernel(x, edge_index, edge_attr, W_pre, b_pre, W_post1, b_post1,
           W_post2, b_post2):
    src = edge_index[0]
    dst = edge_index[1]
    psrc, pdst = _pre_mm(x, W_pre[0:D, :], W_pre[D:2 * D, :])
    q = _edge_mm(edge_attr, W_pre[2 * D:, :], b_pre.reshape(1, D))
    pdst_pad = jnp.pad(pdst, ((0, NPAD - N), (0, 0)))
    s, sq, mx, mn, deg = _sc_aggregate(psrc, pdst_pad, q, src, dst)
    out = _post_mm(x, s[:N], sq[:N], mx[:N], mn[:N],
                   deg[:N].reshape(N, 1), W_post1,
                   b_post1.reshape(1, D), W_post2, b_post2.reshape(1, D))
    return out


# speedup vs baseline: 4.8851x; 1.0019x over previous
import functools
import math

import jax
import jax.numpy as jnp
from jax import lax
from jax.experimental import pallas as pl
from jax.experimental.pallas import tpu as pltpu
from jax.experimental.pallas import tpu_sc as plsc

N = 10000
E = 160000
D = 256
EDGE_DIM = 16
AVG_D_LOG = float(math.log(16.0))
EPS = 1e-5
F32 = jnp.float32

NC, NS, L = 2, 16, 16
NW = NC * NS
BLK = 32
BLK_SHIFT = 5
NBLK = (N + BLK - 1) // BLK
PASSES = (NBLK + NW - 1) // NW
NPAD = NBLK * BLK
CHUNK = 3200
UNROLL = 5
CAPA = 5888
CAPB = 1024
GB = 48
DC = D // L
NEG = -3.0e38
POS = 3.0e38


def _sc_agg_body(psrc_hbm, pdst_hbm, q_hbm, src_hbm, dst_hbm,
                 out_sum, out_sq, out_mx, out_mn, out_deg,
                 src_chunk, dst_chunk, idsA, srcA, dstA,
                 srcB, qidB, lB, srcS, qidS, lS,
                 pdst_buf, psrc0, psrc1, qr0, qr1,
                 acc_sum, acc_sq, acc_mx, acc_mn, deg_buf,
                 hist, offs0, offs1,
                 semp0, semp1, semq0, semq1):
    w = lax.axis_index("s") * NC + lax.axis_index("c")
    iota = lax.iota(jnp.int32, L)
    fz = jnp.zeros((L,), F32)
    vneg = jnp.full((L,), NEG, F32)
    vpos = jnp.full((L,), POS, F32)

    def compact(refs, vals, m, cnt, cap):
        mi = jnp.where(m, 1, 0)
        incl = plsc.cumsum(mi)
        pos = cnt + (incl - mi)
        for ref, v in zip(refs, vals):
            plsc.store_scatter(ref, [pos], v, mask=m)
        return jnp.minimum(cnt + incl[L - 1], cap)

    def chunk_body(ch, cnt):
        pltpu.sync_copy(src_hbm.at[pl.ds(ch * CHUNK, CHUNK)], src_chunk)
        pltpu.sync_copy(dst_hbm.at[pl.ds(ch * CHUNK, CHUNK)], dst_chunk)

        def vreg_body(i, cnt):
            for u in range(UNROLL):
                off = (i * UNROLL + u) * L
                d = dst_chunk[pl.ds(off, L)]
                s = src_chunk[pl.ds(off, L)]
                blk = jnp.right_shift(d, BLK_SHIFT)
                m = jnp.bitwise_and(blk, NW - 1) == w
                eid = (ch * CHUNK + off) + iota
                cnt = compact((idsA, srcA, dstA), (eid, s, d), m, cnt,
                              CAPA - L)
            return cnt

        return lax.fori_loop(0, CHUNK // (L * UNROLL), vreg_body, cnt)

    cntA = lax.fori_loop(0, E // CHUNK, chunk_body, jnp.int32(0))
    nA = (cntA + L - 1) // L

    def pass_body(p, _):
        b = p * NW + w

        @pl.when(b < NBLK)
        def _():
            base = b * BLK

            def init_body(r, _):
                for c in range(DC):
                    sl = pl.ds(c * L, L)
                    acc_sum[r, sl] = fz
                    acc_sq[r, sl] = fz
                    acc_mx[r, sl] = vneg
                    acc_mn[r, sl] = vpos
                return 0

            lax.fori_loop(0, BLK + 1, init_body, 0)

            pltpu.sync_copy(pdst_hbm.at[pl.ds(base, BLK)],
                            pdst_buf.at[pl.ds(0, BLK)])

            def sub_body(i, cnt):
                dsl = dstA[pl.ds(i * L, L)]
                ssl = srcA[pl.ds(i * L, L)]
                esl = idsA[pl.ds(i * L, L)]
                valid = (i * L + iota) < cntA
                m = (jnp.right_shift(dsl, BLK_SHIFT) == b) & valid
                return compact((srcB, qidB, lB), (ssl, esl, dsl - base),
                               m, cnt, CAPB - GB)

            cntB = lax.fori_loop(0, nA, sub_body, jnp.int32(0))

            for t in range(GB // L):
                lB[pl.ds(cntB + t * L, L)] = jnp.full((L,), BLK, jnp.int32)
                srcB[pl.ds(cntB + t * L, L)] = iota
                qidB[pl.ds(cntB + t * L, L)] = iota

            nb = (cntB + GB - 1) // GB
            nv = nb * (GB // L)

            for i in range(BLK + 2):
                hist[i] = jnp.int32(0)

            def hist_body(v, _):
                lvec = lB[pl.ds(v * L, L)]
                for j in range(L):
                    l = lvec[j]
                    hist[l] = hist[l] + 1
                return 0

            lax.fori_loop(0, nv, hist_body, 0)

            run = jnp.int32(0)
            for l in range(BLK + 2):
                offs0[l] = run
                offs1[l] = run
                if l <= BLK:
                    run = run + hist[l]

            def perm_body(v, _):
                lvec = lB[pl.ds(v * L, L)]
                svec = srcB[pl.ds(v * L, L)]
                qvec = qidB[pl.ds(v * L, L)]
                posv = jnp.zeros((L,), jnp.int32)
                for j in range(L):
                    l = lvec[j]
                    pos = offs1[l]
                    offs1[l] = pos + 1
                    posv = jnp.where(iota == j, pos, posv)
                plsc.store_scatter(srcS, [posv], svec)
                plsc.store_scatter(qidS, [posv], qvec)
                plsc.store_scatter(lS, [posv], lvec)
                return 0

            lax.fori_loop(0, nv, perm_body, 0)

            for vi in range(BLK // L):
                dv = jnp.zeros((L,), F32)
                for j in range(L):
                    dv = jnp.where(iota == j,
                                   hist[vi * L + j].astype(F32), dv)
                deg_buf[pl.ds(vi * L, L)] = dv

            rows = ((psrc0, qr0, semp0, semq0), (psrc1, qr1, semp1, semq1))

            def start(t, slot):
                pr, qr, sp, sq_ = rows[slot]
                pltpu.async_copy(psrc_hbm.at[srcS.at[pl.ds(t * GB, GB)]],
                                 pr, sp)
                pltpu.async_copy(q_hbm.at[qidS.at[pl.ds(t * GB, GB)]],
                                 qr, sq_)

            @pl.when(nb > 0)
            def _():
                start(0, 0)

            def batch_body(t, slot):
                pr, qr, sp, sq_ = rows[slot]
                i0 = t * GB
                pltpu.make_async_copy(
                    psrc_hbm.at[srcS.at[pl.ds(i0, GB)]], pr, sp).wait()
                pltpu.make_async_copy(
                    q_hbm.at[qidS.at[pl.ds(i0, GB)]], qr, sq_).wait()

                @pl.when(t + 1 < nb)
                def _():
                    start(t + 1, 1 - slot)

                psrc_rows, q_rows = pr, qr
                l_lo = lS[pl.ds(i0, L)][0]
                l_hi = lS[pl.ds(i0 + GB - L, L)][L - 1]

                def node_body(l, _):
                    o0 = jnp.maximum(offs0[l], i0)
                    o1 = jnp.minimum(offs0[l + 1], i0 + GB)

                    @pl.when(o1 > o0)
                    def _():
                        for g in range(4):
                            pd = [pdst_buf[l, pl.ds((g * 4 + cc) * L, L)]
                                  for cc in range(4)]
                            zz = jnp.zeros((L,), F32)
                            init = ([zz] * 4 + [zz] * 4
                                    + [vneg] * 4 + [vpos] * 4)

                            def jbody(j, carry):
                                r = j - i0
                                out = list(carry)
                                for cc in range(4):
                                    sl = pl.ds((g * 4 + cc) * L, L)
                                    e = jnp.maximum(
                                        psrc_rows[r, sl] + q_rows[r, sl]
                                        + pd[cc], 0.0)
                                    out[cc] = carry[cc] + e
                                    out[4 + cc] = carry[4 + cc] + e * e
                                    out[8 + cc] = jnp.maximum(
                                        carry[8 + cc], e)
                                    out[12 + cc] = jnp.minimum(
                                        carry[12 + cc], e)
                                return tuple(out)

                            res = lax.fori_loop(o0, o1, jbody, tuple(init))
                            for cc in range(4):
                                sl = pl.ds((g * 4 + cc) * L, L)
                                plsc.addupdate(acc_sum.at[l, sl], res[cc])
                                plsc.addupdate(acc_sq.at[l, sl],
                                               res[4 + cc])
                                acc_mx[l, sl] = jnp.maximum(
                                    acc_mx[l, sl], res[8 + cc])
                                acc_mn[l, sl] = jnp.minimum(
                                    acc_mn[l, sl], res[12 + cc])
                    return 0

                lax.fori_loop(l_lo, l_hi + 1, node_body, 0)

            def pair_body(gp, _):
                for s2 in range(2):
                    t = gp * 2 + s2

                    @pl.when(t < nb)
                    def _():
                        batch_body(t, s2)
                return 0

            lax.fori_loop(0, (nb + 1) // 2, pair_body, 0)

            pltpu.sync_copy(acc_sum.at[pl.ds(0, BLK)],
                            out_sum.at[pl.ds(base, BLK)])
            pltpu.sync_copy(acc_sq.at[pl.ds(0, BLK)],
                            out_sq.at[pl.ds(base, BLK)])
            pltpu.sync_copy(acc_mx.at[pl.ds(0, BLK)],
                            out_mx.at[pl.ds(base, BLK)])
            pltpu.sync_copy(acc_mn.at[pl.ds(0, BLK)],
                            out_mn.at[pl.ds(base, BLK)])
            pltpu.sync_copy(deg_buf.at[pl.ds(0, BLK)],
                            out_deg.at[pl.ds(base, BLK)])

        return 0

    lax.fori_loop(0, PASSES, pass_body, 0)


def _sc_aggregate(psrc, pdst_pad, q, src, dst):
    mesh = plsc.VectorSubcoreMesh(core_axis_name="c", subcore_axis_name="s",
                                  num_cores=NC, num_subcores=NS)
    f = pl.kernel(
        _sc_agg_body,
        out_type=(
            jax.ShapeDtypeStruct((NPAD, D), F32),
            jax.ShapeDtypeStruct((NPAD, D), F32),
            jax.ShapeDtypeStruct((NPAD, D), F32),
            jax.ShapeDtypeStruct((NPAD, D), F32),
            jax.ShapeDtypeStruct((NPAD,), F32),
        ),
        mesh=mesh,
        compiler_params=pltpu.CompilerParams(needs_layout_passes=False),
        scratch_types=[
            pltpu.VMEM((CHUNK,), jnp.int32),
            pltpu.VMEM((CHUNK,), jnp.int32),
            pltpu.VMEM((CAPA,), jnp.int32),
            pltpu.VMEM((CAPA,), jnp.int32),
            pltpu.VMEM((CAPA,), jnp.int32),
            pltpu.VMEM((CAPB,), jnp.int32),
            pltpu.VMEM((CAPB,), jnp.int32),
            pltpu.VMEM((CAPB,), jnp.int32),
            pltpu.VMEM((CAPB,), jnp.int32),
            pltpu.VMEM((CAPB,), jnp.int32),
            pltpu.VMEM((CAPB,), jnp.int32),
            pltpu.VMEM((BLK + 1, D), F32),
            pltpu.VMEM((GB, D), F32),
            pltpu.VMEM((GB, D), F32),
            pltpu.VMEM((GB, D), F32),
            pltpu.VMEM((GB, D), F32),
            pltpu.VMEM((BLK + 1, D), F32),
            pltpu.VMEM((BLK + 1, D), F32),
            pltpu.VMEM((BLK + 1, D), F32),
            pltpu.VMEM((BLK + 1, D), F32),
            pltpu.VMEM((BLK,), F32),
            pltpu.SMEM((BLK + 2,), jnp.int32),
            pltpu.SMEM((BLK + 2,), jnp.int32),
            pltpu.SMEM((BLK + 2,), jnp.int32),
            pltpu.SemaphoreType.DMA,
            pltpu.SemaphoreType.DMA,
            pltpu.SemaphoreType.DMA,
            pltpu.SemaphoreType.DMA,
        ],
    )
    return f(psrc, pdst_pad, q, src, dst)


TM = 400
TE = 3200


def _pre_body(xr, w1r, w2r, o1r, o2r):
    xv = xr[...]
    o1r[...] = jnp.dot(xv, w1r[...], preferred_element_type=F32)
    o2r[...] = jnp.dot(xv, w2r[...], preferred_element_type=F32)


def _edge_body(ar, wr, br, qr):
    qr[...] = jnp.dot(ar[...], wr[...], preferred_element_type=F32) + br[...]


def _post_body(xr, sr, qr, mxr, mnr, degr, w1r, b1r, w2r, b2r, outr):
    deg = degr[...]
    degc = jnp.maximum(deg, 1.0)
    inv = 1.0 / degc
    has = deg > 0.0
    logd = jnp.log(deg + 1.0)
    amp = logd * (1.0 / AVG_D_LOG)
    att = AVG_D_LOG / jnp.where(logd > 0.0, logd, 1.0)

    mean = sr[...] * inv
    msq = qr[...] * inv
    var = jnp.maximum(msq - mean * mean, 0.0)
    std = jnp.sqrt(var + EPS)
    zero = jnp.zeros((), F32)
    aggs = (jnp.where(has, mean, zero),
            jnp.where(has, mxr[...], zero),
            jnp.where(has, mnr[...], zero),
            jnp.where(has, std, zero))

    xv = xr[...]
    bf = jnp.bfloat16
    acc = jnp.dot(xv.astype(bf), w1r[0:D, :], preferred_element_type=F32)
    for j, z in enumerate(aggs):
        zb = z.astype(bf)
        y_id = jnp.dot(zb, w1r[(1 + j) * D:(2 + j) * D, :],
                       preferred_element_type=F32)
        y_amp = jnp.dot(zb, w1r[(5 + j) * D:(6 + j) * D, :],
                        preferred_element_type=F32)
        y_att = jnp.dot(zb, w1r[(9 + j) * D:(10 + j) * D, :],
                        preferred_element_type=F32)
        acc = acc + y_id + amp * y_amp + att * y_att
    hidden = jnp.maximum(acc + b1r[...], 0.0)
    outr[...] = (jnp.dot(hidden.astype(bf), w2r[...],
                         preferred_element_type=F32) + b2r[...] + xv)


def _pre_mm(x, w1, w2):
    return pl.pallas_call(
        _pre_body,
        grid=(N // TM,),
        in_specs=[
            pl.BlockSpec((TM, D), lambda i: (i, 0)),
            pl.BlockSpec((D, D), lambda i: (0, 0)),
            pl.BlockSpec((D, D), lambda i: (0, 0)),
        ],
        out_specs=[
            pl.BlockSpec((TM, D), lambda i: (i, 0)),
            pl.BlockSpec((TM, D), lambda i: (i, 0)),
        ],
        out_shape=[
            jax.ShapeDtypeStruct((N, D), F32),
            jax.ShapeDtypeStruct((N, D), F32),
        ],
    )(x, w1, w2)


def _edge_mm(edge_attr, w3, b):
    return pl.pallas_call(
        _edge_body,
        grid=(E // TE,),
        in_specs=[
            pl.BlockSpec((TE, EDGE_DIM), lambda i: (i, 0)),
            pl.BlockSpec((EDGE_DIM, D), lambda i: (0, 0)),
            pl.BlockSpec((1, D), lambda i: (0, 0)),
        ],
        out_specs=pl.BlockSpec((TE, D), lambda i: (i, 0)),
        out_shape=jax.ShapeDtypeStruct((E, D), F32),
    )(edge_attr, w3, b)


def _post_mm(x, s, sq, mx, mn, deg2d, w1, b1, w2, b2):
    return pl.pallas_call(
        _post_body,
        grid=(N // TM,),
        in_specs=[
            pl.BlockSpec((TM, D), lambda i: (i, 0)),
            pl.BlockSpec((TM, D), lambda i: (i, 0)),
            pl.BlockSpec((TM, D), lambda i: (i, 0)),
            pl.BlockSpec((TM, D), lambda i: (i, 0)),
            pl.BlockSpec((TM, D), lambda i: (i, 0)),
            pl.BlockSpec((TM, 1), lambda i: (i, 0)),
            pl.BlockSpec((13 * D, D), lambda i: (0, 0)),
            pl.BlockSpec((1, D), lambda i: (0, 0)),
            pl.BlockSpec((D, D), lambda i: (0, 0)),
            pl.BlockSpec((1, D), lambda i: (0, 0)),
        ],
        out_specs=pl.BlockSpec((TM, D), lambda i: (i, 0)),
        out_shape=jax.ShapeDtypeStruct((N, D), F32),
    )(x, s, sq, mx, mn, deg2d, w1, b1, w2, b2)


def kernel(x, edge_index, edge_attr, W_pre, b_pre, W_post1, b_post1,
           W_post2, b_post2):
    src = edge_index[0]
    dst = edge_index[1]
    psrc, pdst = _pre_mm(x, W_pre[0:D, :], W_pre[D:2 * D, :])
    q = _edge_mm(edge_attr, W_pre[2 * D:, :], b_pre.reshape(1, D))
    pdst_pad = jnp.pad(pdst, ((0, NPAD - N), (0, 0)))
    s, sq, mx, mn, deg = _sc_aggregate(psrc, pdst_pad, q, src, dst)
    out = _post_mm(x, s[:N], sq[:N], mx[:N], mn[:N],
                   deg[:N].reshape(N, 1), W_post1.astype(jnp.bfloat16),
                   b_post1.reshape(1, D), W_post2.astype(jnp.bfloat16),
                   b_post2.reshape(1, D))
    return out


# speedup vs baseline: 4.9541x; 1.0141x over previous
import functools
import math

import jax
import jax.numpy as jnp
from jax import lax
from jax.experimental import pallas as pl
from jax.experimental.pallas import tpu as pltpu
from jax.experimental.pallas import tpu_sc as plsc

N = 10000
E = 160000
D = 256
EDGE_DIM = 16
AVG_D_LOG = float(math.log(16.0))
EPS = 1e-5
F32 = jnp.float32

NC, NS, L = 2, 16, 16
NW = NC * NS
BLK = 32
BLK_SHIFT = 5
NBLK = (N + BLK - 1) // BLK
PASSES = (NBLK + NW - 1) // NW
NPAD = NBLK * BLK
CHUNK = 3200
UNROLL = 5
CAPA = 5888
CAPB = 1024
GB = 48
DC = D // L
NEG = -3.0e38
POS = 3.0e38


def _sc_agg_body(psrc_hbm, pdst_hbm, q_hbm, src_hbm, dst_hbm,
                 out_sum, out_sq, out_mx, out_mn, out_deg,
                 src_chunk, dst_chunk, idsA, srcA,
                 srcB, qidB, lB, srcS, qidS, lS,
                 pdst_buf, psrc0, psrc1, qr0, qr1,
                 acc_sum, acc_sq, acc_mx, acc_mn, deg_buf,
                 hist, offs0, offs1,
                 semp0, semp1, semq0, semq1):
    w = lax.axis_index("s") * NC + lax.axis_index("c")
    iota = lax.iota(jnp.int32, L)
    fz = jnp.zeros((L,), F32)
    vneg = jnp.full((L,), NEG, F32)
    vpos = jnp.full((L,), POS, F32)

    def compact(refs, vals, m, cnt, cap):
        mi = jnp.where(m, 1, 0)
        incl = plsc.cumsum(mi)
        pos = cnt + (incl - mi)
        for ref, v in zip(refs, vals):
            plsc.store_scatter(ref, [pos], v, mask=m)
        pc = plsc.all_reduce_population_count(m)[0]
        return jnp.minimum(cnt + pc, cap)

    def chunk_body(ch, cnt):
        pltpu.sync_copy(src_hbm.at[pl.ds(ch * CHUNK, CHUNK)], src_chunk)
        pltpu.sync_copy(dst_hbm.at[pl.ds(ch * CHUNK, CHUNK)], dst_chunk)

        def vreg_body(i, cnt):
            for u in range(UNROLL):
                off = (i * UNROLL + u) * L
                d = dst_chunk[pl.ds(off, L)]
                s = src_chunk[pl.ds(off, L)]
                blk = jnp.right_shift(d, BLK_SHIFT)
                m = jnp.bitwise_and(blk, NW - 1) == w
                eid = (ch * CHUNK + off) + iota
                pk = jnp.bitwise_or(s, jnp.left_shift(d, 14))
                cnt = compact((idsA, srcA), (eid, pk), m, cnt, CAPA - L)
            return cnt

        return lax.fori_loop(0, CHUNK // (L * UNROLL), vreg_body, cnt)

    cntA = lax.fori_loop(0, E // CHUNK, chunk_body, jnp.int32(0))
    nA = (cntA + L - 1) // L

    def pass_body(p, _):
        b = p * NW + w

        @pl.when(b < NBLK)
        def _():
            base = b * BLK

            def init_body(r, _):
                for c in range(DC):
                    sl = pl.ds(c * L, L)
                    acc_sum[r, sl] = fz
                    acc_sq[r, sl] = fz
                    acc_mx[r, sl] = vneg
                    acc_mn[r, sl] = vpos
                return 0

            lax.fori_loop(0, BLK + 1, init_body, 0)

            pltpu.sync_copy(pdst_hbm.at[pl.ds(base, BLK)],
                            pdst_buf.at[pl.ds(0, BLK)])

            def sub_body(i, cnt):
                pk = srcA[pl.ds(i * L, L)]
                esl = idsA[pl.ds(i * L, L)]
                valid = (i * L + iota) < cntA
                m = (jnp.right_shift(pk, 14 + BLK_SHIFT) == b) & valid
                ssl = jnp.bitwise_and(pk, 16383)
                lsl = jnp.right_shift(pk, 14) - base
                return compact((srcB, qidB, lB), (ssl, esl, lsl),
                               m, cnt, CAPB - GB)

            cntB = lax.fori_loop(0, nA, sub_body, jnp.int32(0))

            for t in range(GB // L):
                lB[pl.ds(cntB + t * L, L)] = jnp.full((L,), BLK, jnp.int32)
                srcB[pl.ds(cntB + t * L, L)] = iota
                qidB[pl.ds(cntB + t * L, L)] = iota

            nb = (cntB + GB - 1) // GB
            nv = nb * (GB // L)

            for i in range(BLK + 2):
                hist[i] = jnp.int32(0)

            def hist_body(v, _):
                lvec = lB[pl.ds(v * L, L)]
                for j in range(L):
                    l = lvec[j]
                    hist[l] = hist[l] + 1
                return 0

            lax.fori_loop(0, nv, hist_body, 0)

            run = jnp.int32(0)
            for l in range(BLK + 2):
                offs0[l] = run
                offs1[l] = run
                if l <= BLK:
                    run = run + hist[l]

            def perm_body(v, _):
                lvec = lB[pl.ds(v * L, L)]
                svec = srcB[pl.ds(v * L, L)]
                qvec = qidB[pl.ds(v * L, L)]
                posv = jnp.zeros((L,), jnp.int32)
                for j in range(L):
                    l = lvec[j]
                    pos = offs1[l]
                    offs1[l] = pos + 1
                    posv = jnp.where(iota == j, pos, posv)
                plsc.store_scatter(srcS, [posv], svec)
                plsc.store_scatter(qidS, [posv], qvec)
                plsc.store_scatter(lS, [posv], lvec)
                return 0

            lax.fori_loop(0, nv, perm_body, 0)

            for vi in range(BLK // L):
                dv = jnp.zeros((L,), F32)
                for j in range(L):
                    dv = jnp.where(iota == j,
                                   hist[vi * L + j].astype(F32), dv)
                deg_buf[pl.ds(vi * L, L)] = dv

            rows = ((psrc0, qr0, semp0, semq0), (psrc1, qr1, semp1, semq1))

            def start(t, slot):
                pr, qr, sp, sq_ = rows[slot]
                pltpu.async_copy(psrc_hbm.at[srcS.at[pl.ds(t * GB, GB)]],
                                 pr, sp)
                pltpu.async_copy(q_hbm.at[qidS.at[pl.ds(t * GB, GB)]],
                                 qr, sq_)

            @pl.when(nb > 0)
            def _():
                start(0, 0)

            def batch_body(t, slot):
                pr, qr, sp, sq_ = rows[slot]
                i0 = t * GB
                pltpu.make_async_copy(
                    psrc_hbm.at[srcS.at[pl.ds(i0, GB)]], pr, sp).wait()
                pltpu.make_async_copy(
                    q_hbm.at[qidS.at[pl.ds(i0, GB)]], qr, sq_).wait()

                @pl.when(t + 1 < nb)
                def _():
                    start(t + 1, 1 - slot)

                psrc_rows, q_rows = pr, qr
                l_lo = lS[pl.ds(i0, L)][0]
                l_hi = lS[pl.ds(i0 + GB - L, L)][L - 1]

                def node_body(l, _):
                    o0 = jnp.maximum(offs0[l], i0)
                    o1 = jnp.minimum(offs0[l + 1], i0 + GB)

                    @pl.when(o1 > o0)
                    def _():
                        for g in range(4):
                            pd = [pdst_buf[l, pl.ds((g * 4 + cc) * L, L)]
                                  for cc in range(4)]
                            zz = jnp.zeros((L,), F32)
                            init = ([zz] * 4 + [zz] * 4
                                    + [vneg] * 4 + [vpos] * 4)

                            def jbody(j, carry):
                                r = j - i0
                                out = list(carry)
                                for cc in range(4):
                                    sl = pl.ds((g * 4 + cc) * L, L)
                                    e = jnp.maximum(
                                        psrc_rows[r, sl] + q_rows[r, sl]
                                        + pd[cc], 0.0)
                                    out[cc] = carry[cc] + e
                                    out[4 + cc] = carry[4 + cc] + e * e
                                    out[8 + cc] = jnp.maximum(
                                        carry[8 + cc], e)
                                    out[12 + cc] = jnp.minimum(
                                        carry[12 + cc], e)
                                return tuple(out)

                            res = lax.fori_loop(o0, o1, jbody, tuple(init))
                            for cc in range(4):
                                sl = pl.ds((g * 4 + cc) * L, L)
                                plsc.addupdate(acc_sum.at[l, sl], res[cc])
                                plsc.addupdate(acc_sq.at[l, sl],
                                               res[4 + cc])
                                acc_mx[l, sl] = jnp.maximum(
                                    acc_mx[l, sl], res[8 + cc])
                                acc_mn[l, sl] = jnp.minimum(
                                    acc_mn[l, sl], res[12 + cc])
                    return 0

                lax.fori_loop(l_lo, l_hi + 1, node_body, 0)

            def pair_body(gp, _):
                for s2 in range(2):
                    t = gp * 2 + s2

                    @pl.when(t < nb)
                    def _():
                        batch_body(t, s2)
                return 0

            lax.fori_loop(0, (nb + 1) // 2, pair_body, 0)

            pltpu.sync_copy(acc_sum.at[pl.ds(0, BLK)],
                            out_sum.at[pl.ds(base, BLK)])
            pltpu.sync_copy(acc_sq.at[pl.ds(0, BLK)],
                            out_sq.at[pl.ds(base, BLK)])
            pltpu.sync_copy(acc_mx.at[pl.ds(0, BLK)],
                            out_mx.at[pl.ds(base, BLK)])
            pltpu.sync_copy(acc_mn.at[pl.ds(0, BLK)],
                            out_mn.at[pl.ds(base, BLK)])
            pltpu.sync_copy(deg_buf.at[pl.ds(0, BLK)],
                            out_deg.at[pl.ds(base, BLK)])

        return 0

    lax.fori_loop(0, PASSES, pass_body, 0)


def _sc_aggregate(psrc, pdst_pad, q, src, dst):
    mesh = plsc.VectorSubcoreMesh(core_axis_name="c", subcore_axis_name="s",
                                  num_cores=NC, num_subcores=NS)
    f = pl.kernel(
        _sc_agg_body,
        out_type=(
            jax.ShapeDtypeStruct((NPAD, D), F32),
            jax.ShapeDtypeStruct((NPAD, D), F32),
            jax.ShapeDtypeStruct((NPAD, D), F32),
            jax.ShapeDtypeStruct((NPAD, D), F32),
            jax.ShapeDtypeStruct((NPAD,), F32),
        ),
        mesh=mesh,
        compiler_params=pltpu.CompilerParams(needs_layout_passes=False),
        scratch_types=[
            pltpu.VMEM((CHUNK,), jnp.int32),
            pltpu.VMEM((CHUNK,), jnp.int32),
            pltpu.VMEM((CAPA,), jnp.int32),
            pltpu.VMEM((CAPA,), jnp.int32),
            pltpu.VMEM((CAPB,), jnp.int32),
            pltpu.VMEM((CAPB,), jnp.int32),
            pltpu.VMEM((CAPB,), jnp.int32),
            pltpu.VMEM((CAPB,), jnp.int32),
            pltpu.VMEM((CAPB,), jnp.int32),
            pltpu.VMEM((CAPB,), jnp.int32),
            pltpu.VMEM((BLK + 1, D), F32),
            pltpu.VMEM((GB, D), F32),
            pltpu.VMEM((GB, D), F32),
            pltpu.VMEM((GB, D), F32),
            pltpu.VMEM((GB, D), F32),
            pltpu.VMEM((BLK + 1, D), F32),
            pltpu.VMEM((BLK + 1, D), F32),
            pltpu.VMEM((BLK + 1, D), F32),
            pltpu.VMEM((BLK + 1, D), F32),
            pltpu.VMEM((BLK,), F32),
            pltpu.SMEM((BLK + 2,), jnp.int32),
            pltpu.SMEM((BLK + 2,), jnp.int32),
            pltpu.SMEM((BLK + 2,), jnp.int32),
            pltpu.SemaphoreType.DMA,
            pltpu.SemaphoreType.DMA,
            pltpu.SemaphoreType.DMA,
            pltpu.SemaphoreType.DMA,
        ],
    )
    return f(psrc, pdst_pad, q, src, dst)


TM = 400
TE = 3200


def _pre_body(xr, w1r, w2r, o1r, o2r):
    xv = xr[...]
    o1r[...] = jnp.dot(xv, w1r[...], preferred_element_type=F32)
    o2r[...] = jnp.dot(xv, w2r[...], preferred_element_type=F32)


def _edge_body(ar, wr, br, qr):
    qr[...] = jnp.dot(ar[...], wr[...], preferred_element_type=F32) + br[...]


def _post_body(xr, sr, qr, mxr, mnr, degr, w1r, b1r, w2r, b2r, outr):
    deg = degr[...]
    degc = jnp.maximum(deg, 1.0)
    inv = 1.0 / degc
    has = deg > 0.0
    logd = jnp.log(deg + 1.0)
    amp = logd * (1.0 / AVG_D_LOG)
    att = AVG_D_LOG / jnp.where(logd > 0.0, logd, 1.0)

    mean = sr[...] * inv
    msq = qr[...] * inv
    var = jnp.maximum(msq - mean * mean, 0.0)
    std = jnp.sqrt(var + EPS)
    zero = jnp.zeros((), F32)
    aggs = (jnp.where(has, mean, zero),
            jnp.where(has, mxr[...], zero),
            jnp.where(has, mnr[...], zero),
            jnp.where(has, std, zero))

    xv = xr[...]
    bf = jnp.bfloat16
    acc = jnp.dot(xv.astype(bf), w1r[0:D, :], preferred_element_type=F32)
    for j, z in enumerate(aggs):
        zb = z.astype(bf)
        y_id = jnp.dot(zb, w1r[(1 + j) * D:(2 + j) * D, :],
                       preferred_element_type=F32)
        y_amp = jnp.dot(zb, w1r[(5 + j) * D:(6 + j) * D, :],
                        preferred_element_type=F32)
        y_att = jnp.dot(zb, w1r[(9 + j) * D:(10 + j) * D, :],
                        preferred_element_type=F32)
        acc = acc + y_id + amp * y_amp + att * y_att
    hidden = jnp.maximum(acc + b1r[...], 0.0)
    outr[...] = (jnp.dot(hidden.astype(bf), w2r[...],
                         preferred_element_type=F32) + b2r[...] + xv)


def _pre_mm(x, w1, w2):
    return pl.pallas_call(
        _pre_body,
        grid=(N // TM,),
        in_specs=[
            pl.BlockSpec((TM, D), lambda i: (i, 0)),
            pl.BlockSpec((D, D), lambda i: (0, 0)),
            pl.BlockSpec((D, D), lambda i: (0, 0)),
        ],
        out_specs=[
            pl.BlockSpec((TM, D), lambda i: (i, 0)),
            pl.BlockSpec((TM, D), lambda i: (i, 0)),
        ],
        out_shape=[
            jax.ShapeDtypeStruct((N, D), F32),
            jax.ShapeDtypeStruct((N, D), F32),
        ],
    )(x, w1, w2)


def _edge_mm(edge_attr, w3, b):
    return pl.pallas_call(
        _edge_body,
        grid=(E // TE,),
        in_specs=[
            pl.BlockSpec((TE, EDGE_DIM), lambda i: (i, 0)),
            pl.BlockSpec((EDGE_DIM, D), lambda i: (0, 0)),
            pl.BlockSpec((1, D), lambda i: (0, 0)),
        ],
        out_specs=pl.BlockSpec((TE, D), lambda i: (i, 0)),
        out_shape=jax.ShapeDtypeStruct((E, D), F32),
    )(edge_attr, w3, b)


def _post_mm(x, s, sq, mx, mn, deg2d, w1, b1, w2, b2):
    return pl.pallas_call(
        _post_body,
        grid=(N // TM,),
        in_specs=[
            pl.BlockSpec((TM, D), lambda i: (i, 0)),
            pl.BlockSpec((TM, D), lambda i: (i, 0)),
            pl.BlockSpec((TM, D), lambda i: (i, 0)),
            pl.BlockSpec((TM, D), lambda i: (i, 0)),
            pl.BlockSpec((TM, D), lambda i: (i, 0)),
            pl.BlockSpec((TM, 1), lambda i: (i, 0)),
            pl.BlockSpec((13 * D, D), lambda i: (0, 0)),
            pl.BlockSpec((1, D), lambda i: (0, 0)),
            pl.BlockSpec((D, D), lambda i: (0, 0)),
            pl.BlockSpec((1, D), lambda i: (0, 0)),
        ],
        out_specs=pl.BlockSpec((TM, D), lambda i: (i, 0)),
        out_shape=jax.ShapeDtypeStruct((N, D), F32),
    )(x, s, sq, mx, mn, deg2d, w1, b1, w2, b2)


def kernel(x, edge_index, edge_attr, W_pre, b_pre, W_post1, b_post1,
           W_post2, b_post2):
    src = edge_index[0]
    dst = edge_index[1]
    psrc, pdst = _pre_mm(x, W_pre[0:D, :], W_pre[D:2 * D, :])
    q = _edge_mm(edge_attr, W_pre[2 * D:, :], b_pre.reshape(1, D))
    pdst_pad = jnp.pad(pdst, ((0, NPAD - N), (0, 0)))
    s, sq, mx, mn, deg = _sc_aggregate(psrc, pdst_pad, q, src, dst)
    out = _post_mm(x, s[:N], sq[:N], mx[:N], mn[:N],
                   deg[:N].reshape(N, 1), W_post1.astype(jnp.bfloat16),
                   b_post1.reshape(1, D), W_post2.astype(jnp.bfloat16),
                   b_post2.reshape(1, D))
    return out


# speedup vs baseline: 5.1904x; 1.0477x over previous
import functools
import math

import jax
import jax.numpy as jnp
from jax import lax
from jax.experimental import pallas as pl
from jax.experimental.pallas import tpu as pltpu
from jax.experimental.pallas import tpu_sc as plsc

N = 10000
E = 160000
D = 256
EDGE_DIM = 16
AVG_D_LOG = float(math.log(16.0))
EPS = 1e-5
F32 = jnp.float32

NC, NS, L = 2, 16, 16
NW = NC * NS
BLK = 32
BLK_SHIFT = 5
NBLK = (N + BLK - 1) // BLK
PASSES = (NBLK + NW - 1) // NW
NPAD = NBLK * BLK
CHUNK = 3200
UNROLL = 5
CAPA = 5888
CAPB = 1024
GB = 48
DC = D // L
NEG = -3.0e38
POS = 3.0e38


def _sc_agg_body(psrc_hbm, pdst_hbm, q_hbm, src_hbm, dst_hbm,
                 out_sum, out_sq, out_mx, out_mn, out_deg,
                 src_chunk, dst_chunk, idsA, srcA,
                 srcB, qidB, lB, srcS, qidS, lS,
                 pdst_buf, psrc0, psrc1, qr0, qr1,
                 acc_sum, acc_sq, acc_mx, acc_mn, deg_buf,
                 hist, offs0, offs1,
                 semp0, semp1, semq0, semq1):
    w = lax.axis_index("s") * NC + lax.axis_index("c")
    iota = lax.iota(jnp.int32, L)
    fz = jnp.zeros((L,), F32)
    vneg = jnp.full((L,), NEG, F32)
    vpos = jnp.full((L,), POS, F32)

    def compact(refs, vals, m, cnt, cap):
        mi = jnp.where(m, 1, 0)
        incl = plsc.cumsum(mi)
        pos = cnt + (incl - mi)
        for ref, v in zip(refs, vals):
            plsc.store_scatter(ref, [pos], v, mask=m)
        pc = plsc.all_reduce_population_count(m)[0]
        return jnp.minimum(cnt + pc, cap)

    NCH = E // CHUNK
    w_off = (w * NCH) // NW

    def chunk_body(ch, cnt):
        cc = lax.rem(ch + w_off, NCH)
        pltpu.sync_copy(src_hbm.at[pl.ds(cc * CHUNK, CHUNK)], src_chunk)
        pltpu.sync_copy(dst_hbm.at[pl.ds(cc * CHUNK, CHUNK)], dst_chunk)

        def vreg_body(i, cnt):
            for u in range(UNROLL):
                off = (i * UNROLL + u) * L
                d = dst_chunk[pl.ds(off, L)]
                s = src_chunk[pl.ds(off, L)]
                blk = jnp.right_shift(d, BLK_SHIFT)
                m = jnp.bitwise_and(blk, NW - 1) == w
                eid = (cc * CHUNK + off) + iota
                pk = jnp.bitwise_or(s, jnp.left_shift(d, 14))
                cnt = compact((idsA, srcA), (eid, pk), m, cnt, CAPA - L)
            return cnt

        return lax.fori_loop(0, CHUNK // (L * UNROLL), vreg_body, cnt)

    cntA = lax.fori_loop(0, NCH, chunk_body, jnp.int32(0))
    nA = (cntA + L - 1) // L

    def pass_body(p, _):
        b = p * NW + w

        @pl.when(b < NBLK)
        def _():
            base = b * BLK

            def init_body(r, _):
                for c in range(DC):
                    sl = pl.ds(c * L, L)
                    acc_sum[r, sl] = fz
                    acc_sq[r, sl] = fz
                    acc_mx[r, sl] = vneg
                    acc_mn[r, sl] = vpos
                return 0

            lax.fori_loop(0, BLK + 1, init_body, 0)

            pltpu.sync_copy(pdst_hbm.at[pl.ds(base, BLK)],
                            pdst_buf.at[pl.ds(0, BLK)])

            def sub_body(i, cnt):
                pk = srcA[pl.ds(i * L, L)]
                esl = idsA[pl.ds(i * L, L)]
                valid = (i * L + iota) < cntA
                m = (jnp.right_shift(pk, 14 + BLK_SHIFT) == b) & valid
                ssl = jnp.bitwise_and(pk, 16383)
                lsl = jnp.right_shift(pk, 14) - base
                return compact((srcB, qidB, lB), (ssl, esl, lsl),
                               m, cnt, CAPB - GB)

            cntB = lax.fori_loop(0, nA, sub_body, jnp.int32(0))

            for t in range(GB // L):
                lB[pl.ds(cntB + t * L, L)] = jnp.full((L,), BLK, jnp.int32)
                srcB[pl.ds(cntB + t * L, L)] = iota
                qidB[pl.ds(cntB + t * L, L)] = iota

            nb = (cntB + GB - 1) // GB
            nv = nb * (GB // L)

            for i in range(BLK + 2):
                hist[i] = jnp.int32(0)

            def hist_body(v, _):
                lvec = lB[pl.ds(v * L, L)]
                for j in range(L):
                    l = lvec[j]
                    hist[l] = hist[l] + 1
                return 0

            lax.fori_loop(0, nv, hist_body, 0)

            run = jnp.int32(0)
            for l in range(BLK + 2):
                offs0[l] = run
                offs1[l] = run
                if l <= BLK:
                    run = run + hist[l]

            def perm_body(v, _):
                lvec = lB[pl.ds(v * L, L)]
                svec = srcB[pl.ds(v * L, L)]
                qvec = qidB[pl.ds(v * L, L)]
                posv = jnp.zeros((L,), jnp.int32)
                for j in range(L):
                    l = lvec[j]
                    pos = offs1[l]
                    offs1[l] = pos + 1
                    posv = jnp.where(iota == j, pos, posv)
                plsc.store_scatter(srcS, [posv], svec)
                plsc.store_scatter(qidS, [posv], qvec)
                plsc.store_scatter(lS, [posv], lvec)
                return 0

            lax.fori_loop(0, nv, perm_body, 0)

            for vi in range(BLK // L):
                dv = jnp.zeros((L,), F32)
                for j in range(L):
                    dv = jnp.where(iota == j,
                                   hist[vi * L + j].astype(F32), dv)
                deg_buf[pl.ds(vi * L, L)] = dv

            rows = ((psrc0, qr0, semp0, semq0), (psrc1, qr1, semp1, semq1))

            def start(t, slot):
                pr, qr, sp, sq_ = rows[slot]
                pltpu.async_copy(psrc_hbm.at[srcS.at[pl.ds(t * GB, GB)]],
                                 pr, sp)
                pltpu.async_copy(q_hbm.at[qidS.at[pl.ds(t * GB, GB)]],
                                 qr, sq_)

            @pl.when(nb > 0)
            def _():
                start(0, 0)

            def batch_body(t, slot):
                pr, qr, sp, sq_ = rows[slot]
                i0 = t * GB
                pltpu.make_async_copy(
                    psrc_hbm.at[srcS.at[pl.ds(i0, GB)]], pr, sp).wait()
                pltpu.make_async_copy(
                    q_hbm.at[qidS.at[pl.ds(i0, GB)]], qr, sq_).wait()

                @pl.when(t + 1 < nb)
                def _():
                    start(t + 1, 1 - slot)

                psrc_rows, q_rows = pr, qr
                l_lo = lS[pl.ds(i0, L)][0]
                l_hi = lS[pl.ds(i0 + GB - L, L)][L - 1]

                def node_body(l, _):
                    o0 = jnp.maximum(offs0[l], i0)
                    o1 = jnp.minimum(offs0[l + 1], i0 + GB)

                    @pl.when(o1 > o0)
                    def _():
                        for g in range(4):
                            pd = [pdst_buf[l, pl.ds((g * 4 + cc) * L, L)]
                                  for cc in range(4)]
                            zz = jnp.zeros((L,), F32)
                            init = ([zz] * 4 + [zz] * 4
                                    + [vneg] * 4 + [vpos] * 4)

                            def jbody(j, carry):
                                r = j - i0
                                out = list(carry)
                                for cc in range(4):
                                    sl = pl.ds((g * 4 + cc) * L, L)
                                    e = jnp.maximum(
                                        psrc_rows[r, sl] + q_rows[r, sl]
                                        + pd[cc], 0.0)
                                    out[cc] = carry[cc] + e
                                    out[4 + cc] = carry[4 + cc] + e * e
                                    out[8 + cc] = jnp.maximum(
                                        carry[8 + cc], e)
                                    out[12 + cc] = jnp.minimum(
                                        carry[12 + cc], e)
                                return tuple(out)

                            res = lax.fori_loop(o0, o1, jbody, tuple(init))
                            for cc in range(4):
                                sl = pl.ds((g * 4 + cc) * L, L)
                                plsc.addupdate(acc_sum.at[l, sl], res[cc])
                                plsc.addupdate(acc_sq.at[l, sl],
                                               res[4 + cc])
                                acc_mx[l, sl] = jnp.maximum(
                                    acc_mx[l, sl], res[8 + cc])
                                acc_mn[l, sl] = jnp.minimum(
                                    acc_mn[l, sl], res[12 + cc])
                    return 0

                lax.fori_loop(l_lo, l_hi + 1, node_body, 0)

            def pair_body(gp, _):
                for s2 in range(2):
                    t = gp * 2 + s2

                    @pl.when(t < nb)
                    def _():
                        batch_body(t, s2)
                return 0

            lax.fori_loop(0, (nb + 1) // 2, pair_body, 0)

            pltpu.sync_copy(acc_sum.at[pl.ds(0, BLK)],
                            out_sum.at[pl.ds(base, BLK)])
            pltpu.sync_copy(acc_sq.at[pl.ds(0, BLK)],
                            out_sq.at[pl.ds(base, BLK)])
            pltpu.sync_copy(acc_mx.at[pl.ds(0, BLK)],
                            out_mx.at[pl.ds(base, BLK)])
            pltpu.sync_copy(acc_mn.at[pl.ds(0, BLK)],
                            out_mn.at[pl.ds(base, BLK)])
            pltpu.sync_copy(deg_buf.at[pl.ds(0, BLK)],
                            out_deg.at[pl.ds(base, BLK)])

        return 0

    lax.fori_loop(0, PASSES, pass_body, 0)


def _sc_aggregate(psrc, pdst_pad, q, src, dst):
    mesh = plsc.VectorSubcoreMesh(core_axis_name="c", subcore_axis_name="s",
                                  num_cores=NC, num_subcores=NS)
    f = pl.kernel(
        _sc_agg_body,
        out_type=(
            jax.ShapeDtypeStruct((NPAD, D), F32),
            jax.ShapeDtypeStruct((NPAD, D), F32),
            jax.ShapeDtypeStruct((NPAD, D), F32),
            jax.ShapeDtypeStruct((NPAD, D), F32),
            jax.ShapeDtypeStruct((NPAD,), F32),
        ),
        mesh=mesh,
        compiler_params=pltpu.CompilerParams(needs_layout_passes=False),
        scratch_types=[
            pltpu.VMEM((CHUNK,), jnp.int32),
            pltpu.VMEM((CHUNK,), jnp.int32),
            pltpu.VMEM((CAPA,), jnp.int32),
            pltpu.VMEM((CAPA,), jnp.int32),
            pltpu.VMEM((CAPB,), jnp.int32),
            pltpu.VMEM((CAPB,), jnp.int32),
            pltpu.VMEM((CAPB,), jnp.int32),
            pltpu.VMEM((CAPB,), jnp.int32),
            pltpu.VMEM((CAPB,), jnp.int32),
            pltpu.VMEM((CAPB,), jnp.int32),
            pltpu.VMEM((BLK + 1, D), F32),
            pltpu.VMEM((GB, D), F32),
            pltpu.VMEM((GB, D), F32),
            pltpu.VMEM((GB, D), F32),
            pltpu.VMEM((GB, D), F32),
            pltpu.VMEM((BLK + 1, D), F32),
            pltpu.VMEM((BLK + 1, D), F32),
            pltpu.VMEM((BLK + 1, D), F32),
            pltpu.VMEM((BLK + 1, D), F32),
            pltpu.VMEM((BLK,), F32),
            pltpu.SMEM((BLK + 2,), jnp.int32),
            pltpu.SMEM((BLK + 2,), jnp.int32),
            pltpu.SMEM((BLK + 2,), jnp.int32),
            pltpu.SemaphoreType.DMA,
            pltpu.SemaphoreType.DMA,
            pltpu.SemaphoreType.DMA,
            pltpu.SemaphoreType.DMA,
        ],
    )
    return f(psrc, pdst_pad, q, src, dst)


TM = 400
TE = 3200


def _pre_body(xr, w1r, w2r, o1r, o2r):
    xv = xr[...]
    o1r[...] = jnp.dot(xv, w1r[...], preferred_element_type=F32)
    o2r[...] = jnp.dot(xv, w2r[...], preferred_element_type=F32)


def _edge_body(ar, wr, br, qr):
    qr[...] = jnp.dot(ar[...], wr[...], preferred_element_type=F32) + br[...]


def _post_body(xr, sr, qr, mxr, mnr, degr, w1r, b1r, w2r, b2r, outr):
    deg = degr[...]
    degc = jnp.maximum(deg, 1.0)
    inv = 1.0 / degc
    has = deg > 0.0
    logd = jnp.log(deg + 1.0)
    amp = logd * (1.0 / AVG_D_LOG)
    att = AVG_D_LOG / jnp.where(logd > 0.0, logd, 1.0)

    mean = sr[...] * inv
    msq = qr[...] * inv
    var = jnp.maximum(msq - mean * mean, 0.0)
    std = jnp.sqrt(var + EPS)
    zero = jnp.zeros((), F32)
    aggs = (jnp.where(has, mean, zero),
            jnp.where(has, mxr[...], zero),
            jnp.where(has, mnr[...], zero),
            jnp.where(has, std, zero))

    xv = xr[...]
    bf = jnp.bfloat16
    acc = jnp.dot(xv.astype(bf), w1r[0:D, :], preferred_element_type=F32)
    for j, z in enumerate(aggs):
        zb = z.astype(bf)
        y_id = jnp.dot(zb, w1r[(1 + j) * D:(2 + j) * D, :],
                       preferred_element_type=F32)
        y_amp = jnp.dot(zb, w1r[(5 + j) * D:(6 + j) * D, :],
                        preferred_element_type=F32)
        y_att = jnp.dot(zb, w1r[(9 + j) * D:(10 + j) * D, :],
                        preferred_element_type=F32)
        acc = acc + y_id + amp * y_amp + att * y_att
    hidden = jnp.maximum(acc + b1r[...], 0.0)
    outr[...] = (jnp.dot(hidden.astype(bf), w2r[...],
                         preferred_element_type=F32) + b2r[...] + xv)


def _pre_mm(x, w1, w2):
    return pl.pallas_call(
        _pre_body,
        grid=(N // TM,),
        in_specs=[
            pl.BlockSpec((TM, D), lambda i: (i, 0)),
            pl.BlockSpec((D, D), lambda i: (0, 0)),
            pl.BlockSpec((D, D), lambda i: (0, 0)),
        ],
        out_specs=[
            pl.BlockSpec((TM, D), lambda i: (i, 0)),
            pl.BlockSpec((TM, D), lambda i: (i, 0)),
        ],
        out_shape=[
            jax.ShapeDtypeStruct((N, D), F32),
            jax.ShapeDtypeStruct((NPAD, D), F32),
        ],
    )(x, w1, w2)


def _edge_mm(edge_attr, w3, b):
    return pl.pallas_call(
        _edge_body,
        grid=(E // TE,),
        in_specs=[
            pl.BlockSpec((TE, EDGE_DIM), lambda i: (i, 0)),
            pl.BlockSpec((EDGE_DIM, D), lambda i: (0, 0)),
            pl.BlockSpec((1, D), lambda i: (0, 0)),
        ],
        out_specs=pl.BlockSpec((TE, D), lambda i: (i, 0)),
        out_shape=jax.ShapeDtypeStruct((E, D), F32),
    )(edge_attr, w3, b)


def _post_mm(x, s, sq, mx, mn, deg2d, w1, b1, w2, b2):
    return pl.pallas_call(
        _post_body,
        grid=(N // TM,),
        in_specs=[
            pl.BlockSpec((TM, D), lambda i: (i, 0)),
            pl.BlockSpec((TM, D), lambda i: (i, 0)),
            pl.BlockSpec((TM, D), lambda i: (i, 0)),
            pl.BlockSpec((TM, D), lambda i: (i, 0)),
            pl.BlockSpec((TM, D), lambda i: (i, 0)),
            pl.BlockSpec((TM, 1), lambda i: (i, 0)),
            pl.BlockSpec((13 * D, D), lambda i: (0, 0)),
            pl.BlockSpec((1, D), lambda i: (0, 0)),
            pl.BlockSpec((D, D), lambda i: (0, 0)),
            pl.BlockSpec((1, D), lambda i: (0, 0)),
        ],
        out_specs=pl.BlockSpec((TM, D), lambda i: (i, 0)),
        out_shape=jax.ShapeDtypeStruct((N, D), F32),
    )(x, s, sq, mx, mn, deg2d, w1, b1, w2, b2)


def kernel(x, edge_index, edge_attr, W_pre, b_pre, W_post1, b_post1,
           W_post2, b_post2):
    src = edge_index[0]
    dst = edge_index[1]
    psrc, pdst_pad = _pre_mm(x, W_pre[0:D, :], W_pre[D:2 * D, :])
    q = _edge_mm(edge_attr, W_pre[2 * D:, :], b_pre.reshape(1, D))
    s, sq, mx, mn, deg = _sc_aggregate(psrc, pdst_pad, q, src, dst)
    out = _post_mm(x, s, sq, mx, mn,
                   deg.reshape(NPAD, 1), W_post1.astype(jnp.bfloat16),
                   b_post1.reshape(1, D), W_post2.astype(jnp.bfloat16),
                   b_post2.reshape(1, D))
    return out


# speedup vs baseline: 6.5073x; 1.2537x over previous
import functools
import math

import jax
import jax.numpy as jnp
from jax import lax
from jax.experimental import pallas as pl
from jax.experimental.pallas import tpu as pltpu
from jax.experimental.pallas import tpu_sc as plsc

N = 10000
E = 160000
D = 256
EDGE_DIM = 16
AVG_D_LOG = float(math.log(16.0))
EPS = 1e-5
F32 = jnp.float32

NC, NS, L = 2, 16, 16
NW = NC * NS
BLK = 32
BLK_SHIFT = 5
NBLK = (N + BLK - 1) // BLK
PASSES = (NBLK + NW - 1) // NW
NPAD = NBLK * BLK
CHUNK = 3200
UNROLL = 5
CAPA = 5888
CAPB = 1024
GB = 48
DC = D // L
NEG = -3.0e38
POS = 3.0e38


def _sc_agg_body(psrc_hbm, pdst_hbm, q_hbm, src_hbm, dst_hbm,
                 out_sum, out_sq, out_mx, out_mn, out_deg,
                 src_chunk, dst_chunk, idsA, srcA,
                 srcB, qidB, lB, srcS, qidS, lS,
                 pdst_buf, psrc0, psrc1, qr0, qr1,
                 acc_sum, acc_sq, acc_mx, acc_mn, deg_buf,
                 hist, offs0, offs1,
                 semp0, semp1, semq0, semq1):
    w = lax.axis_index("s") * NC + lax.axis_index("c")
    iota = lax.iota(jnp.int32, L)
    fz = jnp.zeros((L,), F32)
    vneg = jnp.full((L,), NEG, F32)
    vpos = jnp.full((L,), POS, F32)

    def compact(refs, vals, m, cnt, cap):
        mi = jnp.where(m, 1, 0)
        incl = plsc.cumsum(mi)
        pos = cnt + (incl - mi)
        for ref, v in zip(refs, vals):
            plsc.store_scatter(ref, [pos], v, mask=m)
        pc = plsc.all_reduce_population_count(m)[0]
        return jnp.minimum(cnt + pc, cap)

    NCH = E // CHUNK
    w_off = (w * NCH) // NW

    def chunk_body(ch, cnt):
        cc = lax.rem(ch + w_off, NCH)
        pltpu.sync_copy(src_hbm.at[pl.ds(cc * CHUNK, CHUNK)], src_chunk)
        pltpu.sync_copy(dst_hbm.at[pl.ds(cc * CHUNK, CHUNK)], dst_chunk)

        def vreg_body(i, cnt):
            off = i * L
            d = dst_chunk[pl.ds(off, L)]
            s = src_chunk[pl.ds(off, L)]
            blk = jnp.right_shift(d, BLK_SHIFT)
            m = jnp.bitwise_and(blk, NW - 1) == w
            eid = (cc * CHUNK + off) + iota
            pk = jnp.bitwise_or(s, jnp.left_shift(d, 14))
            return compact((idsA, srcA), (eid, pk), m, cnt, CAPA - L)

        return plsc.parallel_loop(0, CHUNK // L, carry=cnt,
                                  unroll=UNROLL)(vreg_body)

    cntA = lax.fori_loop(0, NCH, chunk_body, jnp.int32(0))
    nA = (cntA + L - 1) // L

    def pass_body(p, _):
        b = p * NW + w

        @pl.when(b < NBLK)
        def _():
            base = b * BLK

            def init_body(r, _):
                for c in range(DC):
                    sl = pl.ds(c * L, L)
                    acc_sum[r, sl] = fz
                    acc_sq[r, sl] = fz
                    acc_mx[r, sl] = vneg
                    acc_mn[r, sl] = vpos
                return 0

            lax.fori_loop(0, BLK + 1, init_body, 0)

            pltpu.sync_copy(pdst_hbm.at[pl.ds(base, BLK)],
                            pdst_buf.at[pl.ds(0, BLK)])

            def sub_body(i, cnt):
                pk = srcA[pl.ds(i * L, L)]
                esl = idsA[pl.ds(i * L, L)]
                valid = (i * L + iota) < cntA
                m = (jnp.right_shift(pk, 14 + BLK_SHIFT) == b) & valid
                ssl = jnp.bitwise_and(pk, 16383)
                lsl = jnp.right_shift(pk, 14) - base
                return compact((srcB, qidB, lB), (ssl, esl, lsl),
                               m, cnt, CAPB - GB)

            cntB = plsc.parallel_loop(0, nA, carry=jnp.int32(0),
                                      unroll=4)(sub_body)

            for t in range(GB // L):
                lB[pl.ds(cntB + t * L, L)] = jnp.full((L,), BLK, jnp.int32)
                srcB[pl.ds(cntB + t * L, L)] = iota
                qidB[pl.ds(cntB + t * L, L)] = iota

            nb = (cntB + GB - 1) // GB
            nv = nb * (GB // L)

            for i in range(BLK + 2):
                hist[i] = jnp.int32(0)

            def hist_body(v, _):
                lvec = lB[pl.ds(v * L, L)]
                for j in range(L):
                    l = lvec[j]
                    hist[l] = hist[l] + 1
                return 0

            lax.fori_loop(0, nv, hist_body, 0)

            run = jnp.int32(0)
            for l in range(BLK + 2):
                offs0[l] = run
                offs1[l] = run
                if l <= BLK:
                    run = run + hist[l]

            def perm_body(v, _):
                lvec = lB[pl.ds(v * L, L)]
                svec = srcB[pl.ds(v * L, L)]
                qvec = qidB[pl.ds(v * L, L)]
                posv = jnp.zeros((L,), jnp.int32)
                for j in range(L):
                    l = lvec[j]
                    pos = offs1[l]
                    offs1[l] = pos + 1
                    posv = jnp.where(iota == j, pos, posv)
                plsc.store_scatter(srcS, [posv], svec)
                plsc.store_scatter(qidS, [posv], qvec)
                plsc.store_scatter(lS, [posv], lvec)
                return 0

            lax.fori_loop(0, nv, perm_body, 0)

            for vi in range(BLK // L):
                dv = jnp.zeros((L,), F32)
                for j in range(L):
                    dv = jnp.where(iota == j,
                                   hist[vi * L + j].astype(F32), dv)
                deg_buf[pl.ds(vi * L, L)] = dv

            rows = ((psrc0, qr0, semp0, semq0), (psrc1, qr1, semp1, semq1))

            def start(t, slot):
                pr, qr, sp, sq_ = rows[slot]
                pltpu.async_copy(psrc_hbm.at[srcS.at[pl.ds(t * GB, GB)]],
                                 pr, sp)
                pltpu.async_copy(q_hbm.at[qidS.at[pl.ds(t * GB, GB)]],
                                 qr, sq_)

            @pl.when(nb > 0)
            def _():
                start(0, 0)

            def batch_body(t, slot):
                pr, qr, sp, sq_ = rows[slot]
                i0 = t * GB
                pltpu.make_async_copy(
                    psrc_hbm.at[srcS.at[pl.ds(i0, GB)]], pr, sp).wait()
                pltpu.make_async_copy(
                    q_hbm.at[qidS.at[pl.ds(i0, GB)]], qr, sq_).wait()

                @pl.when(t + 1 < nb)
                def _():
                    start(t + 1, 1 - slot)

                psrc_rows, q_rows = pr, qr
                l_lo = lS[pl.ds(i0, L)][0]
                l_hi = lS[pl.ds(i0 + GB - L, L)][L - 1]

                def node_body(l, _):
                    o0 = jnp.maximum(offs0[l], i0)
                    o1 = jnp.minimum(offs0[l + 1], i0 + GB)

                    @pl.when(o1 > o0)
                    def _():
                        for g in range(4):
                            pd = [pdst_buf[l, pl.ds((g * 4 + cc) * L, L)]
                                  for cc in range(4)]
                            zz = jnp.zeros((L,), F32)
                            init = ([zz] * 4 + [zz] * 4
                                    + [vneg] * 4 + [vpos] * 4)

                            def jbody(j, carry):
                                r = j - i0
                                out = list(carry)
                                for cc in range(4):
                                    sl = pl.ds((g * 4 + cc) * L, L)
                                    e = jnp.maximum(
                                        psrc_rows[r, sl] + q_rows[r, sl]
                                        + pd[cc], 0.0)
                                    out[cc] = carry[cc] + e
                                    out[4 + cc] = carry[4 + cc] + e * e
                                    out[8 + cc] = jnp.maximum(
                                        carry[8 + cc], e)
                                    out[12 + cc] = jnp.minimum(
                                        carry[12 + cc], e)
                                return tuple(out)

                            res = lax.fori_loop(o0, o1, jbody, tuple(init))
                            for cc in range(4):
                                sl = pl.ds((g * 4 + cc) * L, L)
                                plsc.addupdate(acc_sum.at[l, sl], res[cc])
                                plsc.addupdate(acc_sq.at[l, sl],
                                               res[4 + cc])
                                acc_mx[l, sl] = jnp.maximum(
                                    acc_mx[l, sl], res[8 + cc])
                                acc_mn[l, sl] = jnp.minimum(
                                    acc_mn[l, sl], res[12 + cc])
                    return 0

                lax.fori_loop(l_lo, l_hi + 1, node_body, 0)

            def pair_body(gp, _):
                for s2 in range(2):
                    t = gp * 2 + s2

                    @pl.when(t < nb)
                    def _():
                        batch_body(t, s2)
                return 0

            lax.fori_loop(0, (nb + 1) // 2, pair_body, 0)

            pltpu.sync_copy(acc_sum.at[pl.ds(0, BLK)],
                            out_sum.at[pl.ds(base, BLK)])
            pltpu.sync_copy(acc_sq.at[pl.ds(0, BLK)],
                            out_sq.at[pl.ds(base, BLK)])
            pltpu.sync_copy(acc_mx.at[pl.ds(0, BLK)],
                            out_mx.at[pl.ds(base, BLK)])
            pltpu.sync_copy(acc_mn.at[pl.ds(0, BLK)],
                            out_mn.at[pl.ds(base, BLK)])
            pltpu.sync_copy(deg_buf.at[pl.ds(0, BLK)],
                            out_deg.at[pl.ds(base, BLK)])

        return 0

    lax.fori_loop(0, PASSES, pass_body, 0)


def _sc_aggregate(psrc, pdst_pad, q, src, dst):
    mesh = plsc.VectorSubcoreMesh(core_axis_name="c", subcore_axis_name="s",
                                  num_cores=NC, num_subcores=NS)
    f = pl.kernel(
        _sc_agg_body,
        out_type=(
            jax.ShapeDtypeStruct((NPAD, D), F32),
            jax.ShapeDtypeStruct((NPAD, D), F32),
            jax.ShapeDtypeStruct((NPAD, D), F32),
            jax.ShapeDtypeStruct((NPAD, D), F32),
            jax.ShapeDtypeStruct((NPAD,), F32),
        ),
        mesh=mesh,
        compiler_params=pltpu.CompilerParams(needs_layout_passes=False),
        scratch_types=[
            pltpu.VMEM((CHUNK,), jnp.int32),
            pltpu.VMEM((CHUNK,), jnp.int32),
            pltpu.VMEM((CAPA,), jnp.int32),
            pltpu.VMEM((CAPA,), jnp.int32),
            pltpu.VMEM((CAPB,), jnp.int32),
            pltpu.VMEM((CAPB,), jnp.int32),
            pltpu.VMEM((CAPB,), jnp.int32),
            pltpu.VMEM((CAPB,), jnp.int32),
            pltpu.VMEM((CAPB,), jnp.int32),
            pltpu.VMEM((CAPB,), jnp.int32),
            pltpu.VMEM((BLK + 1, D), F32),
            pltpu.VMEM((GB, D), F32),
            pltpu.VMEM((GB, D), F32),
            pltpu.VMEM((GB, D), F32),
            pltpu.VMEM((GB, D), F32),
            pltpu.VMEM((BLK + 1, D), F32),
            pltpu.VMEM((BLK + 1, D), F32),
            pltpu.VMEM((BLK + 1, D), F32),
            pltpu.VMEM((BLK + 1, D), F32),
            pltpu.VMEM((BLK,), F32),
            pltpu.SMEM((BLK + 2,), jnp.int32),
            pltpu.SMEM((BLK + 2,), jnp.int32),
            pltpu.SMEM((BLK + 2,), jnp.int32),
            pltpu.SemaphoreType.DMA,
            pltpu.SemaphoreType.DMA,
            pltpu.SemaphoreType.DMA,
            pltpu.SemaphoreType.DMA,
        ],
    )
    return f(psrc, pdst_pad, q, src, dst)


TM = 400
TE = 3200


def _pre_body(xr, w1r, w2r, o1r, o2r):
    xv = xr[...]
    o1r[...] = jnp.dot(xv, w1r[...], preferred_element_type=F32)
    o2r[...] = jnp.dot(xv, w2r[...], preferred_element_type=F32)


def _edge_body(ar, wr, br, qr):
    qr[...] = jnp.dot(ar[...], wr[...], preferred_element_type=F32) + br[...]


def _post_body(xr, sr, qr, mxr, mnr, degr, w1r, b1r, w2r, b2r, outr):
    deg = degr[...]
    degc = jnp.maximum(deg, 1.0)
    inv = 1.0 / degc
    has = deg > 0.0
    logd = jnp.log(deg + 1.0)
    amp = logd * (1.0 / AVG_D_LOG)
    att = AVG_D_LOG / jnp.where(logd > 0.0, logd, 1.0)

    mean = sr[...] * inv
    msq = qr[...] * inv
    var = jnp.maximum(msq - mean * mean, 0.0)
    std = jnp.sqrt(var + EPS)
    zero = jnp.zeros((), F32)
    aggs = (jnp.where(has, mean, zero),
            jnp.where(has, mxr[...], zero),
            jnp.where(has, mnr[...], zero),
            jnp.where(has, std, zero))

    xv = xr[...]
    bf = jnp.bfloat16
    acc = jnp.dot(xv.astype(bf), w1r[0:D, :], preferred_element_type=F32)
    for j, z in enumerate(aggs):
        zb = z.astype(bf)
        y_id = jnp.dot(zb, w1r[(1 + j) * D:(2 + j) * D, :],
                       preferred_element_type=F32)
        y_amp = jnp.dot(zb, w1r[(5 + j) * D:(6 + j) * D, :],
                        preferred_element_type=F32)
        y_att = jnp.dot(zb, w1r[(9 + j) * D:(10 + j) * D, :],
                        preferred_element_type=F32)
        acc = acc + y_id + amp * y_amp + att * y_att
    hidden = jnp.maximum(acc + b1r[...], 0.0)
    outr[...] = (jnp.dot(hidden.astype(bf), w2r[...],
                         preferred_element_type=F32) + b2r[...] + xv)


def _pre_mm(x, w1, w2):
    return pl.pallas_call(
        _pre_body,
        grid=(N // TM,),
        in_specs=[
            pl.BlockSpec((TM, D), lambda i: (i, 0)),
            pl.BlockSpec((D, D), lambda i: (0, 0)),
            pl.BlockSpec((D, D), lambda i: (0, 0)),
        ],
        out_specs=[
            pl.BlockSpec((TM, D), lambda i: (i, 0)),
            pl.BlockSpec((TM, D), lambda i: (i, 0)),
        ],
        out_shape=[
            jax.ShapeDtypeStruct((N, D), F32),
            jax.ShapeDtypeStruct((NPAD, D), F32),
        ],
    )(x, w1, w2)


def _edge_mm(edge_attr, w3, b):
    return pl.pallas_call(
        _edge_body,
        grid=(E // TE,),
        in_specs=[
            pl.BlockSpec((TE, EDGE_DIM), lambda i: (i, 0)),
            pl.BlockSpec((EDGE_DIM, D), lambda i: (0, 0)),
            pl.BlockSpec((1, D), lambda i: (0, 0)),
        ],
        out_specs=pl.BlockSpec((TE, D), lambda i: (i, 0)),
        out_shape=jax.ShapeDtypeStruct((E, D), F32),
    )(edge_attr, w3, b)


def _post_mm(x, s, sq, mx, mn, deg2d, w1, b1, w2, b2):
    return pl.pallas_call(
        _post_body,
        grid=(N // TM,),
        in_specs=[
            pl.BlockSpec((TM, D), lambda i: (i, 0)),
            pl.BlockSpec((TM, D), lambda i: (i, 0)),
            pl.BlockSpec((TM, D), lambda i: (i, 0)),
            pl.BlockSpec((TM, D), lambda i: (i, 0)),
            pl.BlockSpec((TM, D), lambda i: (i, 0)),
            pl.BlockSpec((TM, 1), lambda i: (i, 0)),
            pl.BlockSpec((13 * D, D), lambda i: (0, 0)),
            pl.BlockSpec((1, D), lambda i: (0, 0)),
            pl.BlockSpec((D, D), lambda i: (0, 0)),
            pl.BlockSpec((1, D), lambda i: (0, 0)),
        ],
        out_specs=pl.BlockSpec((TM, D), lambda i: (i, 0)),
        out_shape=jax.ShapeDtypeStruct((N, D), F32),
    )(x, s, sq, mx, mn, deg2d, w1, b1, w2, b2)


def kernel(x, edge_index, edge_attr, W_pre, b_pre, W_post1, b_post1,
           W_post2, b_post2):
    src = edge_index[0]
    dst = edge_index[1]
    psrc, pdst_pad = _pre_mm(x, W_pre[0:D, :], W_pre[D:2 * D, :])
    q = _edge_mm(edge_attr, W_pre[2 * D:, :], b_pre.reshape(1, D))
    s, sq, mx, mn, deg = _sc_aggregate(psrc, pdst_pad, q, src, dst)
    out = _post_mm(x, s, sq, mx, mn,
                   deg.reshape(NPAD, 1), W_post1.astype(jnp.bfloat16),
                   b_post1.reshape(1, D), W_post2.astype(jnp.bfloat16),
                   b_post2.reshape(1, D))
    return out
